# Initial kernel scaffold; baseline (speedup 1.0000x reference)
import jax
import jax.numpy as jnp
from jax.experimental import pallas as pl


def kernel(x, edge_index, batch, global_feat, Wl1, bl1, Wr1, br1, att1, bias1, Wl2, bl2, Wr2, br2, att2, bias2, gamma1, beta1, gamma2, beta2, fc1_w, fc1_b, fc2_w, fc2_b):
    raise NotImplementedError("write your pallas kernel here")


# speedup vs baseline: 1.0019x; 1.0019x over previous
import functools
import jax
import jax.numpy as jnp
from jax.experimental import pallas as pl
from jax.experimental.pallas import tpu as pltpu

N = 10000
E = 640000
H = 4
G = 64


def _leaky(x):
    return jnp.where(x >= 0, x, 0.2 * x)


def _matmul2_kernel(x_ref, wl_ref, bl_ref, wr_ref, br_ref, xl_ref, xr_ref):
    x = x_ref[...]
    xl_ref[...] = jnp.dot(x, wl_ref[...], preferred_element_type=jnp.float32) + bl_ref[...]
    xr_ref[...] = jnp.dot(x, wr_ref[...], preferred_element_type=jnp.float32) + br_ref[...]


def _proj2(x, Wl, bl, Wr, br, row_block=2000):
    n, f = x.shape
    k = Wl.shape[1]
    grid = (n // row_block,)
    out_shape = [jax.ShapeDtypeStruct((n, k), jnp.float32)] * 2
    return pl.pallas_call(
        _matmul2_kernel,
        grid=grid,
        in_specs=[
            pl.BlockSpec((row_block, f), lambda i: (i, 0)),
            pl.BlockSpec((f, k), lambda i: (0, 0)),
            pl.BlockSpec((k,), lambda i: (0,)),
            pl.BlockSpec((f, k), lambda i: (0, 0)),
            pl.BlockSpec((k,), lambda i: (0,)),
        ],
        out_specs=[pl.BlockSpec((row_block, k), lambda i: (i, 0))] * 2,
        out_shape=out_shape,
    )(x, Wl, bl, Wr, br)


def _gat_layer(x, src, dst, Wl, bl, Wr, br, att, bias, heads, out_ch):
    n = x.shape[0]
    xl, xr = _proj2(x, Wl, bl, Wr, br)
    xl = xl.reshape(n, heads, out_ch)
    xr = xr.reshape(n, heads, out_ch)
    e = _leaky(xl[src] + xr[dst])
    logits = (e * att[None, :, :]).sum(-1)
    m = jax.ops.segment_max(logits, dst, num_segments=n)
    m = jnp.where(jnp.isfinite(m), m, 0.0)
    ex = jnp.exp(logits - m[dst])
    denom = jax.ops.segment_sum(ex, dst, num_segments=n)
    alpha = ex / (denom[dst] + 1e-16)
    out = jax.ops.segment_sum(xl[src] * alpha[:, :, None], dst, num_segments=n)
    return out.reshape(n, heads * out_ch) + bias


def _bn_relu(x, gamma, beta):
    mu = x.mean(0)
    var = x.var(0)
    return jax.nn.relu((x - mu) / jnp.sqrt(var + 1e-5) * gamma + beta)


def kernel(x, edge_index, batch, global_feat, Wl1, bl1, Wr1, br1, att1, bias1,
           Wl2, bl2, Wr2, br2, att2, bias2, gamma1, beta1, gamma2, beta2,
           fc1_w, fc1_b, fc2_w, fc2_b):
    src, dst = edge_index[0], edge_index[1]
    h = _gat_layer(x, src, dst, Wl1, bl1, Wr1, br1, att1, bias1, 4, 64)
    h = _bn_relu(h, gamma1, beta1)
    h = _gat_layer(h, src, dst, Wl2, bl2, Wr2, br2, att2, bias2, 4, 128)
    h = _bn_relu(h, gamma2, beta2)
    sums = jax.ops.segment_sum(h, batch, num_segments=G)
    counts = jax.ops.segment_sum(jnp.ones((h.shape[0],), h.dtype), batch, num_segments=G)
    pooled = sums / jnp.maximum(counts, 1.0)[:, None]
    z = jnp.concatenate([pooled, global_feat], axis=1)
    z = jax.nn.relu(z @ fc1_w + fc1_b)
    out = z @ fc2_w + fc2_b
    return out.squeeze()


# speedup vs baseline: 19.5673x; 19.5310x over previous
import functools
import jax
import jax.numpy as jnp
from jax import lax
from jax.experimental import pallas as pl
from jax.experimental.pallas import tpu as pltpu
from jax.experimental.pallas import tpu_sc as plsc

N = 10000
E = 640000
H = 4
G = 64
NC = 2
NS = 16
LANES = 16
EPT = E // NS
B = 160
IDXW = 80
Q = B // IDXW
CH = EPT // B
NP = 10240
DR = NP // 8
DRT = DR // NS
DUMPR = 1000
DUMPT = N // DUMPR


def _lane_shuffle(v, idx):
    return lax.gather(
        v, idx[:, None],
        dimension_numbers=lax.GatherDimensionNumbers(
            offset_dims=(), collapsed_slice_dims=(0,), start_index_map=(0,)),
        slice_sizes=(1,),
        mode=lax.GatherScatterMode.PROMISE_IN_BOUNDS)


def _proj_pair_body(x_ref, wl_ref, bl_ref, wr_ref, br_ref, att_ref,
                    xlt_ref, xrt_ref, m_ref):
    c = pl.program_id(0)
    x = x_ref[...]
    blv = bl_ref[pl.ds(c, 1), :]
    brv = br_ref[pl.ds(c, 1), :]
    xl = jnp.dot(x, wl_ref[...], preferred_element_type=jnp.float32) + blv
    xr = jnp.dot(x, wr_ref[...], preferred_element_type=jnp.float32) + brv
    xlt_ref[...] = xl
    xrt_ref[...] = xr
    a0 = jnp.abs(att_ref[pl.ds(2 * c, 1), :])
    a1 = jnp.abs(att_ref[pl.ds(2 * c + 1, 1), :])
    m0 = (jnp.max(jnp.sum(jnp.abs(xl[:, :64]) * a0, axis=1))
          + jnp.max(jnp.sum(jnp.abs(xr[:, :64]) * a0, axis=1)))
    m1 = (jnp.max(jnp.sum(jnp.abs(xl[:, 64:]) * a1, axis=1))
          + jnp.max(jnp.sum(jnp.abs(xr[:, 64:]) * a1, axis=1)))
    m_ref[...] = jnp.concatenate(
        [jnp.full((4, 128), m0, jnp.float32),
         jnp.full((4, 128), m1, jnp.float32)], axis=0)


def _proj_pair(x, Wl, bl, Wr, br, att):
    n, f = x.shape
    return pl.pallas_call(
        _proj_pair_body,
        grid=(NC,),
        in_specs=[
            pl.BlockSpec((n, f), lambda c: (0, 0)),
            pl.BlockSpec((f, 128), lambda c: (0, c)),
            pl.BlockSpec((NC, 128), lambda c: (0, 0)),
            pl.BlockSpec((f, 128), lambda c: (0, c)),
            pl.BlockSpec((NC, 128), lambda c: (0, 0)),
            pl.BlockSpec((H, 64), lambda c: (0, 0)),
        ],
        out_specs=[
            pl.BlockSpec((n, 128), lambda c: (c, 0)),
            pl.BlockSpec((n, 128), lambda c: (c, 0)),
            pl.BlockSpec((8, 128), lambda c: (c, 0)),
        ],
        out_shape=[
            jax.ShapeDtypeStruct((NC * n, 128), jnp.float32),
            jax.ShapeDtypeStruct((NC * n, 128), jnp.float32),
            jax.ShapeDtypeStruct((NC * 8, 128), jnp.float32),
        ],
    )(x, Wl, bl.reshape(NC, 128), Wr, br.reshape(NC, 128), att)


def _proj_head_body(x_ref, wl_ref, bl_ref, wr_ref, br_ref, att_ref,
                    xlt_ref, xrt_ref, m_ref):
    h = pl.program_id(0)
    x = x_ref[...]
    blv = bl_ref[pl.ds(h, 1), :]
    brv = br_ref[pl.ds(h, 1), :]
    xl = jnp.dot(x, wl_ref[...], preferred_element_type=jnp.float32) + blv
    xr = jnp.dot(x, wr_ref[...], preferred_element_type=jnp.float32) + brv
    xlt_ref[...] = xl
    xrt_ref[...] = xr
    aab = jnp.abs(att_ref[pl.ds(h, 1), :])
    m = (jnp.max(jnp.sum(jnp.abs(xl) * aab, axis=1))
         + jnp.max(jnp.sum(jnp.abs(xr) * aab, axis=1)))
    m_ref[...] = jnp.full((8, 128), m, jnp.float32)


def _proj_head(x, Wl, bl, Wr, br, att):
    n, f = x.shape
    C = 128
    wlh = Wl.reshape(f, H, C).transpose(1, 0, 2).reshape(H * f, C)
    wrh = Wr.reshape(f, H, C).transpose(1, 0, 2).reshape(H * f, C)
    return pl.pallas_call(
        _proj_head_body,
        grid=(H,),
        in_specs=[
            pl.BlockSpec((n, f), lambda h: (0, 0)),
            pl.BlockSpec((f, C), lambda h: (h, 0)),
            pl.BlockSpec((H, C), lambda h: (0, 0)),
            pl.BlockSpec((f, C), lambda h: (h, 0)),
            pl.BlockSpec((H, C), lambda h: (0, 0)),
            pl.BlockSpec((H, C), lambda h: (0, 0)),
        ],
        out_specs=[
            pl.BlockSpec((n, C), lambda h: (h, 0)),
            pl.BlockSpec((n, C), lambda h: (h, 0)),
            pl.BlockSpec((8, 128), lambda h: (h, 0)),
        ],
        out_shape=[
            jax.ShapeDtypeStruct((H * n, C), jnp.float32),
            jax.ShapeDtypeStruct((H * n, C), jnp.float32),
            jax.ShapeDtypeStruct((8 * H, 128), jnp.float32),
        ],
    )(x, wlh, bl.reshape(H, C), wrh, br.reshape(H, C), att)


def _dncols(denom_dump):
    d = denom_dump.reshape(NC, DR, 128)[:, :, :16].reshape(NC, NP, 2)
    return d[:, :N, :].transpose(1, 0, 2).reshape(N, H)


def _sel_col(dn, h):
    msk = (lax.broadcasted_iota(jnp.int32, dn.shape, 1) == h).astype(dn.dtype)
    return jnp.sum(dn * msk, axis=1, keepdims=True)


def _norm_pair_body(h_ref, dn_ref, o_ref):
    c = pl.program_id(0)
    hv = h_ref[...]
    dn = dn_ref[...]
    d0 = _sel_col(dn, 2 * c)
    d1 = _sel_col(dn, 2 * c + 1)
    o_ref[...] = jnp.concatenate(
        [hv[:, :64] / (d0 + 1e-16), hv[:, 64:] / (d1 + 1e-16)], axis=1)


def _norm_pair(out1, dn):
    return pl.pallas_call(
        _norm_pair_body,
        grid=(NC,),
        in_specs=[pl.BlockSpec((N, 128), lambda c: (c, 0)),
                  pl.BlockSpec((N, H), lambda c: (0, 0))],
        out_specs=pl.BlockSpec((N, 128), lambda c: (c, 0)),
        out_shape=jax.ShapeDtypeStruct((NC * N, 128), jnp.float32),
    )(out1, dn)


def _norm_head_body(h_ref, dn_ref, o_ref):
    h = pl.program_id(0)
    o_ref[...] = h_ref[...] / (_sel_col(dn_ref[...], h) + 1e-16)


def _norm_head(out2, dn):
    return pl.pallas_call(
        _norm_head_body,
        grid=(H,),
        in_specs=[pl.BlockSpec((N, 128), lambda h: (h, 0)),
                  pl.BlockSpec((N, H), lambda h: (0, 0))],
        out_specs=pl.BlockSpec((N, 128), lambda h: (h, 0)),
        out_shape=jax.ShapeDtypeStruct((H * N, 128), jnp.float32),
    )(out2, dn)


def _bn_relu_body(h_ref, bias_ref, g_ref, b_ref, o_ref):
    hv = h_ref[...] + bias_ref[...]
    mu = jnp.mean(hv, axis=0)
    var = jnp.mean((hv - mu) ** 2, axis=0)
    o_ref[...] = jnp.maximum(
        (hv - mu) / jnp.sqrt(var + 1e-5) * g_ref[...] + b_ref[...], 0.0)


def _bn_relu(h, bias, gamma, beta):
    n, k = h.shape
    return pl.pallas_call(
        _bn_relu_body,
        grid=(k // 128,),
        in_specs=[
            pl.BlockSpec((n, 128), lambda j: (0, j)),
            pl.BlockSpec((128,), lambda j: (j,)),
            pl.BlockSpec((128,), lambda j: (j,)),
            pl.BlockSpec((128,), lambda j: (j,)),
        ],
        out_specs=pl.BlockSpec((n, 128), lambda j: (0, j)),
        out_shape=jax.ShapeDtypeStruct(h.shape, jnp.float32),
    )(h, bias, gamma, beta)


def _final_body(h_ref, batch_ref, gf_ref, w1a_ref, w1b_ref, b1_ref, w2_ref,
                b2_ref, o_ref):
    hv = h_ref[...]
    bat = batch_ref[...]
    gids = lax.broadcasted_iota(jnp.int32, (G, N), 0)
    P = jnp.where(bat == gids, 1.0, 0.0)
    counts = jnp.sum(P, axis=1)
    sums = jnp.dot(P, hv, preferred_element_type=jnp.float32)
    pooled = sums / jnp.maximum(counts, 1.0)[:, None]
    z = (jnp.dot(pooled, w1a_ref[...], preferred_element_type=jnp.float32)
         + jnp.dot(gf_ref[...], w1b_ref[...], preferred_element_type=jnp.float32)
         + b1_ref[...])
    z = jnp.maximum(z, 0.0)
    o_ref[...] = jnp.dot(z, w2_ref[...], preferred_element_type=jnp.float32) + b2_ref[...]


def _pool_mlp(h, batch, global_feat, fc1_w, fc1_b, fc2_w, fc2_b):
    k = h.shape[1]
    return pl.pallas_call(
        _final_body,
        out_shape=jax.ShapeDtypeStruct((G, 1), jnp.float32),
    )(h, batch.reshape(1, N).astype(jnp.int32), global_feat,
      fc1_w[:k], fc1_w[k:], fc1_b, fc2_w, fc2_b)


def _sc_mesh():
    return plsc.VectorSubcoreMesh(core_axis_name="c", subcore_axis_name="s")


def _build_idx(idx2d, flat_v, off, shift=0):
    for q in range(Q):
        for k in range(IDXW // LANES):
            sl = pl.ds(q * IDXW + k * LANES, LANES)
            v = flat_v[sl]
            if shift:
                v = lax.shift_right_logical(v, shift)
            idx2d[q, pl.ds(k * LANES, LANES)] = v + off


def _gather_rows(table, idx2d, buf, sem):
    cps = [pltpu.async_copy(table.at[idx2d.at[q]],
                            buf.at[pl.ds(q * IDXW, IDXW)], sem)
           for q in range(Q)]
    for cp in cps:
        cp.wait()


def _edge_logits(xl_buf, xr_buf, att_v, m_v, ex_buf, att_off, m_row, col0, C):
    m16 = m_v[m_row, pl.ds(0, LANES)]
    attcs = [att_v[pl.ds(att_off + cb * LANES, LANES)]
             for cb in range(C // LANES)]
    lanes = lax.iota(jnp.int32, LANES)
    bfly = [lanes ^ (1 << kk) for kk in range(4)]

    def grp(g, _):
        lvec = jnp.zeros((LANES,), jnp.float32)
        for j in range(LANES):
            e = g * LANES + j
            acc = jnp.zeros((LANES,), jnp.float32)
            for cb in range(C // LANES):
                csl = pl.ds(col0 + cb * LANES, LANES)
                sv = xl_buf[e, csl] + xr_buf[e, csl]
                acc = acc + jnp.maximum(sv, 0.2 * sv) * attcs[cb]
            for p in bfly:
                acc = acc + _lane_shuffle(acc, p)
            lvec = jnp.where(lanes == j, acc, lvec)
        ex_buf[pl.ds(g * LANES, LANES)] = jnp.exp(lvec - m16)
        return 0

    lax.fori_loop(0, B // LANES, grp, 0)


def _scatter_denom(oh_buf, ex_bufs, dst_v, sidx, denom_sh, lanes):
    def oh(g, _):
        sl = pl.ds(g * LANES, LANES)
        exs = [b[sl] for b in ex_bufs]
        col0 = (dst_v[sl] & 7) * 2
        for j in range(LANES):
            cj = col0[j]
            v = jnp.zeros((LANES,), jnp.float32)
            for hl in range(len(ex_bufs)):
                v = jnp.where(lanes == cj + hl, exs[hl][j], v)
            oh_buf[g * LANES + j, pl.ds(0, LANES)] = v
        return 0

    lax.fori_loop(0, B // LANES, oh, 0)
    for q in range(Q):
        pltpu.sync_copy(oh_buf.at[pl.ds(q * IDXW, IDXW)],
                        denom_sh.at[sidx.at[q]], add=True)


def _make_pass1_pair():
    C = 64

    @functools.partial(
        pl.kernel,
        out_type=[
            jax.ShapeDtypeStruct((H * E,), jnp.float32),
            jax.ShapeDtypeStruct((NC * DR, 128), jnp.float32),
        ],
        mesh=_sc_mesh(),
        scratch_types=[
            pltpu.VMEM((B,), jnp.int32),
            pltpu.VMEM((B,), jnp.int32),
            pltpu.VMEM((Q, IDXW), jnp.int32),
            pltpu.VMEM((Q, IDXW), jnp.int32),
            pltpu.VMEM((Q, IDXW), jnp.int32),
            pltpu.VMEM((B, 128), jnp.float32),
            pltpu.VMEM((B, 128), jnp.float32),
            pltpu.VMEM((B, 128), jnp.float32),
            pltpu.VMEM((B,), jnp.float32),
            pltpu.VMEM((B,), jnp.float32),
            pltpu.VMEM((H * C,), jnp.float32),
            pltpu.VMEM((NC * 8, 128), jnp.float32),
            pltpu.VMEM_SHARED((DR, 128), jnp.float32),
            pltpu.SemaphoreType.DMA,
            pltpu.SemaphoreType.DMA,
        ],
    )
    def p1(xlt, xrt, srch, dsth, atth, mh, zrowh, exh, denomdh,
           src_v, dst_v, gidx, didx, sidx, xl_buf, xr_buf, oh_buf, ex0, ex1,
           att_v, m_v, denom_sh, sem1, sem2):
        c = lax.axis_index("c")
        s = lax.axis_index("s")
        pltpu.sync_copy(atth, att_v)
        pltpu.sync_copy(mh, m_v)
        pltpu.sync_copy(zrowh.at[pl.ds(0, B)], oh_buf)
        pltpu.sync_copy(zrowh.at[pl.ds(s * DRT, DRT)],
                        denom_sh.at[pl.ds(s * DRT, DRT)])
        plsc.subcore_barrier()
        tile_base = s * EPT
        rowoff = c * N
        lanes = lax.iota(jnp.int32, LANES)

        def chunk_body(i, _):
            base = tile_base + i * B
            pltpu.sync_copy(srch.at[pl.ds(base, B)], src_v)
            pltpu.sync_copy(dsth.at[pl.ds(base, B)], dst_v)
            _build_idx(gidx, src_v, rowoff)
            _build_idx(didx, dst_v, rowoff)
            _build_idx(sidx, dst_v, 0, shift=3)
            _gather_rows(xlt, gidx, xl_buf, sem1)
            _gather_rows(xrt, didx, xr_buf, sem2)
            for hl, exb in ((0, ex0), (1, ex1)):
                hg = 2 * c + hl
                _edge_logits(xl_buf, xr_buf, att_v, m_v, exb,
                             hg * C, c * 8 + hl * 4, hl * C, C)
                pltpu.sync_copy(exb, exh.at[pl.ds(hg * E + base, B)])
            _scatter_denom(oh_buf, (ex0, ex1), dst_v, sidx, denom_sh, lanes)
            return 0

        lax.fori_loop(0, CH, chunk_body, 0)
        plsc.subcore_barrier()
        pltpu.sync_copy(denom_sh.at[pl.ds(s * DRT, DRT)],
                        denomdh.at[pl.ds(c * DR + s * DRT, DRT)])

    return p1


def _make_pass2_pair():
    C = 64

    @functools.partial(
        pl.kernel,
        out_type=jax.ShapeDtypeStruct((NC * N, 128), jnp.float32),
        mesh=_sc_mesh(),
        scratch_types=[
            pltpu.VMEM((B,), jnp.int32),
            pltpu.VMEM((B,), jnp.int32),
            pltpu.VMEM((Q, IDXW), jnp.int32),
            pltpu.VMEM((Q, IDXW), jnp.int32),
            pltpu.VMEM((B, 128), jnp.float32),
            pltpu.VMEM((B,), jnp.float32),
            pltpu.VMEM((B,), jnp.float32),
            pltpu.VMEM_SHARED((N, 128), jnp.float32),
            pltpu.SemaphoreType.DMA,
        ],
    )
    def p2(xlt, srch, dsth, exh, zrowh, outh,
           src_v, dst_v, gidx, sidx, xl_buf, ex0, ex1, out_sh, sem):
        c = lax.axis_index("c")
        s = lax.axis_index("s")
        tile_base = s * EPT
        rowoff = c * N

        @pl.when(s < DUMPT)
        def _():
            pltpu.sync_copy(zrowh.at[pl.ds(s * DUMPR, DUMPR)],
                            out_sh.at[pl.ds(s * DUMPR, DUMPR)])

        plsc.subcore_barrier()

        def chunk_body(i, _):
            base = tile_base + i * B
            pltpu.sync_copy(srch.at[pl.ds(base, B)], src_v)
            pltpu.sync_copy(dsth.at[pl.ds(base, B)], dst_v)
            _build_idx(gidx, src_v, rowoff)
            _build_idx(sidx, dst_v, 0)
            _gather_rows(xlt, gidx, xl_buf, sem)
            pltpu.sync_copy(exh.at[pl.ds((2 * c) * E + base, B)], ex0)
            pltpu.sync_copy(exh.at[pl.ds((2 * c + 1) * E + base, B)], ex1)

            def grp(g, _):
                sl = pl.ds(g * LANES, LANES)
                a0 = ex0[sl]
                a1 = ex1[sl]
                for j in range(LANES):
                    e = g * LANES + j
                    aj0 = a0[j]
                    aj1 = a1[j]
                    for cb in range(C // LANES):
                        csl = pl.ds(cb * LANES, LANES)
                        xl_buf[e, csl] = xl_buf[e, csl] * aj0
                    for cb in range(C // LANES):
                        csl = pl.ds(C + cb * LANES, LANES)
                        xl_buf[e, csl] = xl_buf[e, csl] * aj1
                return 0

            lax.fori_loop(0, B // LANES, grp, 0)
            for q in range(Q):
                pltpu.sync_copy(xl_buf.at[pl.ds(q * IDXW, IDXW)],
                                out_sh.at[sidx.at[q]], add=True)
            return 0

        lax.fori_loop(0, CH, chunk_body, 0)
        plsc.subcore_barrier()

        @pl.when(s < DUMPT)
        def _():
            pltpu.sync_copy(out_sh.at[pl.ds(s * DUMPR, DUMPR)],
                            outh.at[pl.ds(rowoff + s * DUMPR, DUMPR)])

    return p2


def _make_pass1_head():
    C = 128

    @functools.partial(
        pl.kernel,
        out_type=[
            jax.ShapeDtypeStruct((H * E,), jnp.float32),
            jax.ShapeDtypeStruct((NC * DR, 128), jnp.float32),
        ],
        mesh=_sc_mesh(),
        scratch_types=[
            pltpu.VMEM((B,), jnp.int32),
            pltpu.VMEM((B,), jnp.int32),
            pltpu.VMEM((Q, IDXW), jnp.int32),
            pltpu.VMEM((Q, IDXW), jnp.int32),
            pltpu.VMEM((Q, IDXW), jnp.int32),
            pltpu.VMEM((B, 128), jnp.float32),
            pltpu.VMEM((B, 128), jnp.float32),
            pltpu.VMEM((B, 128), jnp.float32),
            pltpu.VMEM((B,), jnp.float32),
            pltpu.VMEM((H * C,), jnp.float32),
            pltpu.VMEM((8 * H, 128), jnp.float32),
            pltpu.VMEM_SHARED((DR, 128), jnp.float32),
            pltpu.SemaphoreType.DMA,
            pltpu.SemaphoreType.DMA,
        ],
    )
    def p1(xlt, xrt, srch, dsth, atth, mh, zrowh, exh, denomdh,
           src_v, dst_v, gidx, didx, sidx, xl_buf, xr_buf, oh_buf, ex_buf,
           att_v, m_v, denom_sh, sem1, sem2):
        c = lax.axis_index("c")
        s = lax.axis_index("s")
        pltpu.sync_copy(atth, att_v)
        pltpu.sync_copy(mh, m_v)
        pltpu.sync_copy(zrowh.at[pl.ds(0, B)], oh_buf)
        pltpu.sync_copy(zrowh.at[pl.ds(s * DRT, DRT)],
                        denom_sh.at[pl.ds(s * DRT, DRT)])
        plsc.subcore_barrier()
        tile_base = s * EPT
        lanes = lax.iota(jnp.int32, LANES)

        for hl in range(2):
            hg = 2 * c + hl
            hoff = hg * N

            def chunk_body(i, _, hl=hl, hg=hg, hoff=hoff):
                base = tile_base + i * B
                pltpu.sync_copy(srch.at[pl.ds(base, B)], src_v)
                pltpu.sync_copy(dsth.at[pl.ds(base, B)], dst_v)
                _build_idx(gidx, src_v, hoff)
                _build_idx(didx, dst_v, hoff)
                _build_idx(sidx, dst_v, 0, shift=3)
                _gather_rows(xlt, gidx, xl_buf, sem1)
                _gather_rows(xrt, didx, xr_buf, sem2)
                _edge_logits(xl_buf, xr_buf, att_v, m_v, ex_buf,
                             hg * C, hg * 8, 0, C)
                pltpu.sync_copy(ex_buf, exh.at[pl.ds(hg * E + base, B)])

                def oh(g, _):
                    sl = pl.ds(g * LANES, LANES)
                    exs = ex_buf[sl]
                    col16 = (dst_v[sl] & 7) * 2 + hl
                    for j in range(LANES):
                        v = jnp.where(lanes == col16[j], exs[j], 0.0)
                        oh_buf[g * LANES + j, pl.ds(0, LANES)] = v
                    return 0

                lax.fori_loop(0, B // LANES, oh, 0)
                for q in range(Q):
                    pltpu.sync_copy(oh_buf.at[pl.ds(q * IDXW, IDXW)],
                                    denom_sh.at[sidx.at[q]], add=True)
                return 0

            lax.fori_loop(0, CH, chunk_body, 0)

        plsc.subcore_barrier()
        pltpu.sync_copy(denom_sh.at[pl.ds(s * DRT, DRT)],
                        denomdh.at[pl.ds(c * DR + s * DRT, DRT)])

    return p1


def _make_pass2_head():
    C = 128

    @functools.partial(
        pl.kernel,
        out_type=jax.ShapeDtypeStruct((H * N, 128), jnp.float32),
        mesh=_sc_mesh(),
        scratch_types=[
            pltpu.VMEM((B,), jnp.int32),
            pltpu.VMEM((B,), jnp.int32),
            pltpu.VMEM((Q, IDXW), jnp.int32),
            pltpu.VMEM((Q, IDXW), jnp.int32),
            pltpu.VMEM((B, 128), jnp.float32),
            pltpu.VMEM((B,), jnp.float32),
            pltpu.VMEM_SHARED((N, 128), jnp.float32),
            pltpu.SemaphoreType.DMA,
        ],
    )
    def p2(xlt, srch, dsth, exh, zrowh, outh,
           src_v, dst_v, gidx, sidx, xl_buf, ex_buf, out_sh, sem):
        c = lax.axis_index("c")
        s = lax.axis_index("s")
        tile_base = s * EPT
        for hl in range(2):
            hg = 2 * c + hl
            hoff = hg * N

            @pl.when(s < DUMPT)
            def _():
                pltpu.sync_copy(zrowh.at[pl.ds(s * DUMPR, DUMPR)],
                                out_sh.at[pl.ds(s * DUMPR, DUMPR)])

            plsc.subcore_barrier()

            def chunk_body(i, _, hg=hg, hoff=hoff):
                base = tile_base + i * B
                pltpu.sync_copy(srch.at[pl.ds(base, B)], src_v)
                pltpu.sync_copy(dsth.at[pl.ds(base, B)], dst_v)
                _build_idx(gidx, src_v, hoff)
                _build_idx(sidx, dst_v, 0)
                _gather_rows(xlt, gidx, xl_buf, sem)
                pltpu.sync_copy(exh.at[pl.ds(hg * E + base, B)], ex_buf)

                def grp(g, _):
                    sl = pl.ds(g * LANES, LANES)
                    a16 = ex_buf[sl]
                    for j in range(LANES):
                        e = g * LANES + j
                        aj = a16[j]
                        for cb in range(C // LANES):
                            csl = pl.ds(cb * LANES, LANES)
                            xl_buf[e, csl] = xl_buf[e, csl] * aj
                    return 0

                lax.fori_loop(0, B // LANES, grp, 0)
                for q in range(Q):
                    pltpu.sync_copy(xl_buf.at[pl.ds(q * IDXW, IDXW)],
                                    out_sh.at[sidx.at[q]], add=True)
                return 0

            lax.fori_loop(0, CH, chunk_body, 0)
            plsc.subcore_barrier()

            @pl.when(s < DUMPT)
            def _(hoff=hoff):
                pltpu.sync_copy(out_sh.at[pl.ds(s * DUMPR, DUMPR)],
                                outh.at[pl.ds(hoff + s * DUMPR, DUMPR)])

            plsc.subcore_barrier()

    return p2


_P1_PAIR = _make_pass1_pair()
_P2_PAIR = _make_pass2_pair()
_P1_HEAD = _make_pass1_head()
_P2_HEAD = _make_pass2_head()


def kernel(x, edge_index, batch, global_feat, Wl1, bl1, Wr1, br1, att1, bias1,
           Wl2, bl2, Wr2, br2, att2, bias2, gamma1, beta1, gamma2, beta2,
           fc1_w, fc1_b, fc2_w, fc2_b):
    src = edge_index[0].astype(jnp.int32)
    dst = edge_index[1].astype(jnp.int32)
    zrow = jnp.zeros((NP, 128), jnp.float32)

    xlt1, xrt1, m1 = _proj_pair(x, Wl1, bl1, Wr1, br1, att1)
    ex1, dd1 = _P1_PAIR(xlt1, xrt1, src, dst, att1.reshape(-1), m1, zrow)
    dn1 = _dncols(dd1)
    out1 = _P2_PAIR(xlt1, src, dst, ex1, zrow)
    out1 = _norm_pair(out1, dn1)
    h = out1.reshape(NC, N, 2, 64).transpose(1, 0, 2, 3).reshape(N, 256)
    h = _bn_relu(h, bias1, gamma1, beta1)

    xlt2, xrt2, m2 = _proj_head(h, Wl2, bl2, Wr2, br2, att2)
    ex2, dd2 = _P1_HEAD(xlt2, xrt2, src, dst, att2.reshape(-1), m2, zrow)
    dn2 = _dncols(dd2)
    out2 = _P2_HEAD(xlt2, src, dst, ex2, zrow)
    out2 = _norm_head(out2, dn2)
    h = out2.reshape(H, N, 128).transpose(1, 0, 2).reshape(N, 512)
    h = _bn_relu(h, bias2, gamma2, beta2)

    out = _pool_mlp(h, batch, global_feat, fc1_w, fc1_b, fc2_w, fc2_b)
    return out.reshape(G)


# speedup vs baseline: 21.3677x; 1.0920x over previous
import functools
import jax
import jax.numpy as jnp
from jax import lax
from jax.experimental import pallas as pl
from jax.experimental.pallas import tpu as pltpu
from jax.experimental.pallas import tpu_sc as plsc

N = 10000
E = 640000
H = 4
G = 64
NC = 2
NS = 16
LANES = 16
EPT = E // NS
B = 160
B2 = 320
IDXW = 80
Q = B // IDXW
Q2 = B2 // IDXW
CH = EPT // B
CH2 = EPT // B2
NP = 10240
DR = NP // 8
DRT = DR // NS
DUMPR = 1000
DUMPT = N // DUMPR


def _lane_shuffle(v, idx):
    return lax.gather(
        v, idx[:, None],
        dimension_numbers=lax.GatherDimensionNumbers(
            offset_dims=(), collapsed_slice_dims=(0,), start_index_map=(0,)),
        slice_sizes=(1,),
        mode=lax.GatherScatterMode.PROMISE_IN_BOUNDS)


def _proj_pair_body(x_ref, wl_ref, bl_ref, wr_ref, br_ref, att_ref,
                    xlt_ref, xrt_ref, m_ref):
    c = pl.program_id(0)
    x = x_ref[...]
    blv = bl_ref[pl.ds(c, 1), :]
    brv = br_ref[pl.ds(c, 1), :]
    xl = jnp.dot(x, wl_ref[...], preferred_element_type=jnp.float32) + blv
    xr = jnp.dot(x, wr_ref[...], preferred_element_type=jnp.float32) + brv
    xlt_ref[...] = xl
    xrt_ref[...] = xr
    a0 = jnp.abs(att_ref[pl.ds(2 * c, 1), :])
    a1 = jnp.abs(att_ref[pl.ds(2 * c + 1, 1), :])
    m0 = (jnp.max(jnp.sum(jnp.abs(xl[:, :64]) * a0, axis=1))
          + jnp.max(jnp.sum(jnp.abs(xr[:, :64]) * a0, axis=1)))
    m1 = (jnp.max(jnp.sum(jnp.abs(xl[:, 64:]) * a1, axis=1))
          + jnp.max(jnp.sum(jnp.abs(xr[:, 64:]) * a1, axis=1)))
    m_ref[...] = jnp.concatenate(
        [jnp.full((4, 128), m0, jnp.float32),
         jnp.full((4, 128), m1, jnp.float32)], axis=0)


def _proj_pair(x, Wl, bl, Wr, br, att):
    n, f = x.shape
    return pl.pallas_call(
        _proj_pair_body,
        grid=(NC,),
        in_specs=[
            pl.BlockSpec((n, f), lambda c: (0, 0)),
            pl.BlockSpec((f, 128), lambda c: (0, c)),
            pl.BlockSpec((NC, 128), lambda c: (0, 0)),
            pl.BlockSpec((f, 128), lambda c: (0, c)),
            pl.BlockSpec((NC, 128), lambda c: (0, 0)),
            pl.BlockSpec((H, 64), lambda c: (0, 0)),
        ],
        out_specs=[
            pl.BlockSpec((n, 128), lambda c: (c, 0)),
            pl.BlockSpec((n, 128), lambda c: (c, 0)),
            pl.BlockSpec((8, 128), lambda c: (c, 0)),
        ],
        out_shape=[
            jax.ShapeDtypeStruct((NC * n, 128), jnp.float32),
            jax.ShapeDtypeStruct((NC * n, 128), jnp.float32),
            jax.ShapeDtypeStruct((NC * 8, 128), jnp.float32),
        ],
    )(x, Wl, bl.reshape(NC, 128), Wr, br.reshape(NC, 128), att)


def _proj_head_body(x_ref, wl_ref, bl_ref, wr_ref, br_ref, att_ref,
                    xlt_ref, xrt_ref, m_ref):
    h = pl.program_id(0)
    x = x_ref[...]
    blv = bl_ref[pl.ds(h, 1), :]
    brv = br_ref[pl.ds(h, 1), :]
    xl = jnp.dot(x, wl_ref[...], preferred_element_type=jnp.float32) + blv
    xr = jnp.dot(x, wr_ref[...], preferred_element_type=jnp.float32) + brv
    xlt_ref[...] = xl
    xrt_ref[...] = xr
    aab = jnp.abs(att_ref[pl.ds(h, 1), :])
    m = (jnp.max(jnp.sum(jnp.abs(xl) * aab, axis=1))
         + jnp.max(jnp.sum(jnp.abs(xr) * aab, axis=1)))
    m_ref[...] = jnp.full((8, 128), m, jnp.float32)


def _proj_head(x, Wl, bl, Wr, br, att):
    n, f = x.shape
    C = 128
    wlh = Wl.reshape(f, H, C).transpose(1, 0, 2).reshape(H * f, C)
    wrh = Wr.reshape(f, H, C).transpose(1, 0, 2).reshape(H * f, C)
    return pl.pallas_call(
        _proj_head_body,
        grid=(H,),
        in_specs=[
            pl.BlockSpec((n, f), lambda h: (0, 0)),
            pl.BlockSpec((f, C), lambda h: (h, 0)),
            pl.BlockSpec((H, C), lambda h: (0, 0)),
            pl.BlockSpec((f, C), lambda h: (h, 0)),
            pl.BlockSpec((H, C), lambda h: (0, 0)),
            pl.BlockSpec((H, C), lambda h: (0, 0)),
        ],
        out_specs=[
            pl.BlockSpec((n, C), lambda h: (h, 0)),
            pl.BlockSpec((n, C), lambda h: (h, 0)),
            pl.BlockSpec((8, 128), lambda h: (h, 0)),
        ],
        out_shape=[
            jax.ShapeDtypeStruct((H * n, C), jnp.float32),
            jax.ShapeDtypeStruct((H * n, C), jnp.float32),
            jax.ShapeDtypeStruct((8 * H, 128), jnp.float32),
        ],
    )(x, wlh, bl.reshape(H, C), wrh, br.reshape(H, C), att)


def _dncols(denom_dump):
    d = denom_dump.reshape(NC, DR, 128)[:, :, :16].reshape(NC, NP, 2)
    return d[:, :N, :].transpose(1, 0, 2).reshape(N, H)


def _sel_col(dn, h):
    msk = (lax.broadcasted_iota(jnp.int32, dn.shape, 1) == h).astype(dn.dtype)
    return jnp.sum(dn * msk, axis=1, keepdims=True)


def _norm_pair_body(h_ref, dn_ref, o_ref):
    c = pl.program_id(0)
    hv = h_ref[...]
    dn = dn_ref[...]
    d0 = _sel_col(dn, 2 * c)
    d1 = _sel_col(dn, 2 * c + 1)
    o_ref[...] = jnp.concatenate(
        [hv[:, :64] / (d0 + 1e-16), hv[:, 64:] / (d1 + 1e-16)], axis=1)


def _norm_pair(out1, dn):
    return pl.pallas_call(
        _norm_pair_body,
        grid=(NC,),
        in_specs=[pl.BlockSpec((N, 128), lambda c: (c, 0)),
                  pl.BlockSpec((N, H), lambda c: (0, 0))],
        out_specs=pl.BlockSpec((N, 128), lambda c: (c, 0)),
        out_shape=jax.ShapeDtypeStruct((NC * N, 128), jnp.float32),
    )(out1, dn)


def _norm_head_body(h_ref, dn_ref, o_ref):
    h = pl.program_id(0)
    o_ref[...] = h_ref[...] / (_sel_col(dn_ref[...], h) + 1e-16)


def _norm_head(out2, dn):
    return pl.pallas_call(
        _norm_head_body,
        grid=(H,),
        in_specs=[pl.BlockSpec((N, 128), lambda h: (h, 0)),
                  pl.BlockSpec((N, H), lambda h: (0, 0))],
        out_specs=pl.BlockSpec((N, 128), lambda h: (h, 0)),
        out_shape=jax.ShapeDtypeStruct((H * N, 128), jnp.float32),
    )(out2, dn)


def _bn_relu_body(h_ref, bias_ref, g_ref, b_ref, o_ref):
    hv = h_ref[...] + bias_ref[...]
    mu = jnp.mean(hv, axis=0)
    var = jnp.mean((hv - mu) ** 2, axis=0)
    o_ref[...] = jnp.maximum(
        (hv - mu) / jnp.sqrt(var + 1e-5) * g_ref[...] + b_ref[...], 0.0)


def _bn_relu(h, bias, gamma, beta):
    n, k = h.shape
    return pl.pallas_call(
        _bn_relu_body,
        grid=(k // 128,),
        in_specs=[
            pl.BlockSpec((n, 128), lambda j: (0, j)),
            pl.BlockSpec((128,), lambda j: (j,)),
            pl.BlockSpec((128,), lambda j: (j,)),
            pl.BlockSpec((128,), lambda j: (j,)),
        ],
        out_specs=pl.BlockSpec((n, 128), lambda j: (0, j)),
        out_shape=jax.ShapeDtypeStruct(h.shape, jnp.float32),
    )(h, bias, gamma, beta)


def _final_body(h_ref, batch_ref, gf_ref, w1a_ref, w1b_ref, b1_ref, w2_ref,
                b2_ref, o_ref):
    hv = h_ref[...]
    bat = batch_ref[...]
    gids = lax.broadcasted_iota(jnp.int32, (G, N), 0)
    P = jnp.where(bat == gids, 1.0, 0.0)
    counts = jnp.sum(P, axis=1)
    sums = jnp.dot(P, hv, preferred_element_type=jnp.float32)
    pooled = sums / jnp.maximum(counts, 1.0)[:, None]
    z = (jnp.dot(pooled, w1a_ref[...], preferred_element_type=jnp.float32)
         + jnp.dot(gf_ref[...], w1b_ref[...], preferred_element_type=jnp.float32)
         + b1_ref[...])
    z = jnp.maximum(z, 0.0)
    o_ref[...] = jnp.dot(z, w2_ref[...], preferred_element_type=jnp.float32) + b2_ref[...]


def _pool_mlp(h, batch, global_feat, fc1_w, fc1_b, fc2_w, fc2_b):
    k = h.shape[1]
    return pl.pallas_call(
        _final_body,
        out_shape=jax.ShapeDtypeStruct((G, 1), jnp.float32),
    )(h, batch.reshape(1, N).astype(jnp.int32), global_feat,
      fc1_w[:k], fc1_w[k:], fc1_b, fc2_w, fc2_b)


def _sc_mesh():
    return plsc.VectorSubcoreMesh(core_axis_name="c", subcore_axis_name="s")


def _build_idx(idx2d, flat_v, off, shift=0):
    for q in range(idx2d.shape[0]):
        for k in range(IDXW // LANES):
            sl = pl.ds(q * IDXW + k * LANES, LANES)
            v = flat_v[sl]
            if shift:
                v = lax.shift_right_logical(v, shift)
            idx2d[q, pl.ds(k * LANES, LANES)] = v + off


def _gather_rows(table, idx2d, buf, sem):
    cps = [pltpu.async_copy(table.at[idx2d.at[q]],
                            buf.at[pl.ds(q * IDXW, IDXW)], sem)
           for q in range(idx2d.shape[0])]
    for cp in cps:
        cp.wait()


def _edge_logits(xl_buf, xr_buf, att_v, m_v, ex_buf, att_off, m_row, col0, C):
    m16 = m_v[m_row, pl.ds(0, LANES)]
    attcs = [att_v[pl.ds(att_off + cb * LANES, LANES)]
             for cb in range(C // LANES)]
    lanes = lax.iota(jnp.int32, LANES)
    bfly = [lanes ^ (1 << kk) for kk in range(4)]

    def grp(g, _):
        lvec = jnp.zeros((LANES,), jnp.float32)
        for j in range(LANES):
            e = g * LANES + j
            acc = jnp.zeros((LANES,), jnp.float32)
            for cb in range(C // LANES):
                csl = pl.ds(col0 + cb * LANES, LANES)
                sv = xl_buf[e, csl] + xr_buf[e, csl]
                acc = acc + jnp.maximum(sv, 0.2 * sv) * attcs[cb]
            for p in bfly:
                acc = acc + _lane_shuffle(acc, p)
            lvec = jnp.where(lanes == j, acc, lvec)
        ex_buf[pl.ds(g * LANES, LANES)] = jnp.exp(lvec - m16)
        return 0

    lax.fori_loop(0, B // LANES, grp, 0)


def _scatter_denom(oh_buf, ex_bufs, dst_v, sidx, denom_sh, lanes):
    def oh(g, _):
        sl = pl.ds(g * LANES, LANES)
        exs = [b[sl] for b in ex_bufs]
        col0 = (dst_v[sl] & 7) * 2
        for j in range(LANES):
            cj = col0[j]
            v = jnp.zeros((LANES,), jnp.float32)
            for hl in range(len(ex_bufs)):
                v = jnp.where(lanes == cj + hl, exs[hl][j], v)
            oh_buf[g * LANES + j, pl.ds(0, LANES)] = v
        return 0

    lax.fori_loop(0, B // LANES, oh, 0)
    for q in range(Q):
        pltpu.sync_copy(oh_buf.at[pl.ds(q * IDXW, IDXW)],
                        denom_sh.at[sidx.at[q]], add=True)


def _make_pass1_pair():
    C = 64

    @functools.partial(
        pl.kernel,
        out_type=[
            jax.ShapeDtypeStruct((H * E,), jnp.float32),
            jax.ShapeDtypeStruct((NC * DR, 128), jnp.float32),
        ],
        mesh=_sc_mesh(),
        scratch_types=[
            pltpu.VMEM((B,), jnp.int32),
            pltpu.VMEM((B,), jnp.int32),
            pltpu.VMEM((Q, IDXW), jnp.int32),
            pltpu.VMEM((Q, IDXW), jnp.int32),
            pltpu.VMEM((Q, IDXW), jnp.int32),
            pltpu.VMEM((B, 128), jnp.float32),
            pltpu.VMEM((B, 128), jnp.float32),
            pltpu.VMEM((B, 128), jnp.float32),
            pltpu.VMEM((B,), jnp.float32),
            pltpu.VMEM((B,), jnp.float32),
            pltpu.VMEM((H * C,), jnp.float32),
            pltpu.VMEM((NC * 8, 128), jnp.float32),
            pltpu.VMEM_SHARED((DR, 128), jnp.float32),
            pltpu.SemaphoreType.DMA,
            pltpu.SemaphoreType.DMA,
        ],
    )
    def p1(xlt, xrt, srch, dsth, atth, mh, zrowh, exh, denomdh,
           src_v, dst_v, gidx, didx, sidx, xl_buf, xr_buf, oh_buf, ex0, ex1,
           att_v, m_v, denom_sh, sem1, sem2):
        c = lax.axis_index("c")
        s = lax.axis_index("s")
        pltpu.sync_copy(atth, att_v)
        pltpu.sync_copy(mh, m_v)
        pltpu.sync_copy(zrowh.at[pl.ds(0, B)], oh_buf)
        pltpu.sync_copy(zrowh.at[pl.ds(s * DRT, DRT)],
                        denom_sh.at[pl.ds(s * DRT, DRT)])
        plsc.subcore_barrier()
        tile_base = s * EPT
        rowoff = c * N
        lanes = lax.iota(jnp.int32, LANES)

        def chunk_body(i, _):
            base = tile_base + i * B
            pltpu.sync_copy(srch.at[pl.ds(base, B)], src_v)
            pltpu.sync_copy(dsth.at[pl.ds(base, B)], dst_v)
            _build_idx(gidx, src_v, rowoff)
            _build_idx(didx, dst_v, rowoff)
            _build_idx(sidx, dst_v, 0, shift=3)
            _gather_rows(xlt, gidx, xl_buf, sem1)
            _gather_rows(xrt, didx, xr_buf, sem2)
            for hl, exb in ((0, ex0), (1, ex1)):
                hg = 2 * c + hl
                _edge_logits(xl_buf, xr_buf, att_v, m_v, exb,
                             hg * C, c * 8 + hl * 4, hl * C, C)
                pltpu.sync_copy(exb, exh.at[pl.ds(hg * E + base, B)])
            _scatter_denom(oh_buf, (ex0, ex1), dst_v, sidx, denom_sh, lanes)
            return 0

        lax.fori_loop(0, CH, chunk_body, 0)
        plsc.subcore_barrier()
        pltpu.sync_copy(denom_sh.at[pl.ds(s * DRT, DRT)],
                        denomdh.at[pl.ds(c * DR + s * DRT, DRT)])

    return p1


def _make_pass2_pair():
    C = 64

    @functools.partial(
        pl.kernel,
        out_type=jax.ShapeDtypeStruct((NC * N, 128), jnp.float32),
        mesh=_sc_mesh(),
        scratch_types=[
            pltpu.VMEM((B2,), jnp.int32),
            pltpu.VMEM((B2,), jnp.int32),
            pltpu.VMEM((Q2, IDXW), jnp.int32),
            pltpu.VMEM((Q2, IDXW), jnp.int32),
            pltpu.VMEM((B2, 128), jnp.float32),
            pltpu.VMEM((B2,), jnp.float32),
            pltpu.VMEM((B2,), jnp.float32),
            pltpu.VMEM_SHARED((N, 128), jnp.float32),
            pltpu.SemaphoreType.DMA,
        ],
    )
    def p2(xlt, srch, dsth, exh, zrowh, outh,
           src_v, dst_v, gidx, sidx, xl_buf, ex0, ex1, out_sh, sem):
        c = lax.axis_index("c")
        s = lax.axis_index("s")
        tile_base = s * EPT
        rowoff = c * N

        @pl.when(s < DUMPT)
        def _():
            pltpu.sync_copy(zrowh.at[pl.ds(s * DUMPR, DUMPR)],
                            out_sh.at[pl.ds(s * DUMPR, DUMPR)])

        plsc.subcore_barrier()

        def chunk_body(i, _):
            base = tile_base + i * B2
            pltpu.sync_copy(srch.at[pl.ds(base, B2)], src_v)
            pltpu.sync_copy(dsth.at[pl.ds(base, B2)], dst_v)
            _build_idx(gidx, src_v, rowoff)
            _build_idx(sidx, dst_v, 0)
            _gather_rows(xlt, gidx, xl_buf, sem)
            pltpu.sync_copy(exh.at[pl.ds((2 * c) * E + base, B2)], ex0)
            pltpu.sync_copy(exh.at[pl.ds((2 * c + 1) * E + base, B2)], ex1)

            def grp(g, _):
                sl = pl.ds(g * LANES, LANES)
                a0 = ex0[sl]
                a1 = ex1[sl]
                for j in range(LANES):
                    e = g * LANES + j
                    aj0 = a0[j]
                    aj1 = a1[j]
                    for cb in range(C // LANES):
                        csl = pl.ds(cb * LANES, LANES)
                        xl_buf[e, csl] = xl_buf[e, csl] * aj0
                    for cb in range(C // LANES):
                        csl = pl.ds(C + cb * LANES, LANES)
                        xl_buf[e, csl] = xl_buf[e, csl] * aj1
                return 0

            lax.fori_loop(0, B2 // LANES, grp, 0)
            for q in range(Q2):
                pltpu.sync_copy(xl_buf.at[pl.ds(q * IDXW, IDXW)],
                                out_sh.at[sidx.at[q]], add=True)
            return 0

        lax.fori_loop(0, CH2, chunk_body, 0)
        plsc.subcore_barrier()

        @pl.when(s < DUMPT)
        def _():
            pltpu.sync_copy(out_sh.at[pl.ds(s * DUMPR, DUMPR)],
                            outh.at[pl.ds(rowoff + s * DUMPR, DUMPR)])

    return p2


def _make_pass1_head():
    C = 128

    @functools.partial(
        pl.kernel,
        out_type=[
            jax.ShapeDtypeStruct((H * E,), jnp.float32),
            jax.ShapeDtypeStruct((NC * DR, 128), jnp.float32),
        ],
        mesh=_sc_mesh(),
        scratch_types=[
            pltpu.VMEM((B,), jnp.int32),
            pltpu.VMEM((B,), jnp.int32),
            pltpu.VMEM((Q, IDXW), jnp.int32),
            pltpu.VMEM((Q, IDXW), jnp.int32),
            pltpu.VMEM((Q, IDXW), jnp.int32),
            pltpu.VMEM((B, 128), jnp.float32),
            pltpu.VMEM((B, 128), jnp.float32),
            pltpu.VMEM((B, 128), jnp.float32),
            pltpu.VMEM((B,), jnp.float32),
            pltpu.VMEM((H * C,), jnp.float32),
            pltpu.VMEM((8 * H, 128), jnp.float32),
            pltpu.VMEM_SHARED((DR, 128), jnp.float32),
            pltpu.SemaphoreType.DMA,
            pltpu.SemaphoreType.DMA,
        ],
    )
    def p1(xlt, xrt, srch, dsth, atth, mh, zrowh, exh, denomdh,
           src_v, dst_v, gidx, didx, sidx, xl_buf, xr_buf, oh_buf, ex_buf,
           att_v, m_v, denom_sh, sem1, sem2):
        c = lax.axis_index("c")
        s = lax.axis_index("s")
        pltpu.sync_copy(atth, att_v)
        pltpu.sync_copy(mh, m_v)
        pltpu.sync_copy(zrowh.at[pl.ds(0, B)], oh_buf)
        pltpu.sync_copy(zrowh.at[pl.ds(s * DRT, DRT)],
                        denom_sh.at[pl.ds(s * DRT, DRT)])
        plsc.subcore_barrier()
        tile_base = s * EPT
        lanes = lax.iota(jnp.int32, LANES)

        for hl in range(2):
            hg = 2 * c + hl
            hoff = hg * N

            def chunk_body(i, _, hl=hl, hg=hg, hoff=hoff):
                base = tile_base + i * B
                pltpu.sync_copy(srch.at[pl.ds(base, B)], src_v)
                pltpu.sync_copy(dsth.at[pl.ds(base, B)], dst_v)
                _build_idx(gidx, src_v, hoff)
                _build_idx(didx, dst_v, hoff)
                _build_idx(sidx, dst_v, 0, shift=3)
                _gather_rows(xlt, gidx, xl_buf, sem1)
                _gather_rows(xrt, didx, xr_buf, sem2)
                _edge_logits(xl_buf, xr_buf, att_v, m_v, ex_buf,
                             hg * C, hg * 8, 0, C)
                pltpu.sync_copy(ex_buf, exh.at[pl.ds(hg * E + base, B)])

                def oh(g, _):
                    sl = pl.ds(g * LANES, LANES)
                    exs = ex_buf[sl]
                    col16 = (dst_v[sl] & 7) * 2 + hl
                    for j in range(LANES):
                        v = jnp.where(lanes == col16[j], exs[j], 0.0)
                        oh_buf[g * LANES + j, pl.ds(0, LANES)] = v
                    return 0

                lax.fori_loop(0, B // LANES, oh, 0)
                for q in range(Q):
                    pltpu.sync_copy(oh_buf.at[pl.ds(q * IDXW, IDXW)],
                                    denom_sh.at[sidx.at[q]], add=True)
                return 0

            lax.fori_loop(0, CH, chunk_body, 0)

        plsc.subcore_barrier()
        pltpu.sync_copy(denom_sh.at[pl.ds(s * DRT, DRT)],
                        denomdh.at[pl.ds(c * DR + s * DRT, DRT)])

    return p1


def _make_pass2_head():
    C = 128

    @functools.partial(
        pl.kernel,
        out_type=jax.ShapeDtypeStruct((H * N, 128), jnp.float32),
        mesh=_sc_mesh(),
        scratch_types=[
            pltpu.VMEM((B2,), jnp.int32),
            pltpu.VMEM((B2,), jnp.int32),
            pltpu.VMEM((Q2, IDXW), jnp.int32),
            pltpu.VMEM((Q2, IDXW), jnp.int32),
            pltpu.VMEM((B2, 128), jnp.float32),
            pltpu.VMEM((B2,), jnp.float32),
            pltpu.VMEM_SHARED((N, 128), jnp.float32),
            pltpu.SemaphoreType.DMA,
        ],
    )
    def p2(xlt, srch, dsth, exh, zrowh, outh,
           src_v, dst_v, gidx, sidx, xl_buf, ex_buf, out_sh, sem):
        c = lax.axis_index("c")
        s = lax.axis_index("s")
        tile_base = s * EPT
        for hl in range(2):
            hg = 2 * c + hl
            hoff = hg * N

            @pl.when(s < DUMPT)
            def _():
                pltpu.sync_copy(zrowh.at[pl.ds(s * DUMPR, DUMPR)],
                                out_sh.at[pl.ds(s * DUMPR, DUMPR)])

            plsc.subcore_barrier()

            def chunk_body(i, _, hg=hg, hoff=hoff):
                base = tile_base + i * B2
                pltpu.sync_copy(srch.at[pl.ds(base, B2)], src_v)
                pltpu.sync_copy(dsth.at[pl.ds(base, B2)], dst_v)
                _build_idx(gidx, src_v, hoff)
                _build_idx(sidx, dst_v, 0)
                _gather_rows(xlt, gidx, xl_buf, sem)
                pltpu.sync_copy(exh.at[pl.ds(hg * E + base, B2)], ex_buf)

                def grp(g, _):
                    sl = pl.ds(g * LANES, LANES)
                    a16 = ex_buf[sl]
                    for j in range(LANES):
                        e = g * LANES + j
                        aj = a16[j]
                        for cb in range(C // LANES):
                            csl = pl.ds(cb * LANES, LANES)
                            xl_buf[e, csl] = xl_buf[e, csl] * aj
                    return 0

                lax.fori_loop(0, B2 // LANES, grp, 0)
                for q in range(Q2):
                    pltpu.sync_copy(xl_buf.at[pl.ds(q * IDXW, IDXW)],
                                    out_sh.at[sidx.at[q]], add=True)
                return 0

            lax.fori_loop(0, CH2, chunk_body, 0)
            plsc.subcore_barrier()

            @pl.when(s < DUMPT)
            def _(hoff=hoff):
                pltpu.sync_copy(out_sh.at[pl.ds(s * DUMPR, DUMPR)],
                                outh.at[pl.ds(hoff + s * DUMPR, DUMPR)])

            plsc.subcore_barrier()

    return p2


_P1_PAIR = _make_pass1_pair()
_P2_PAIR = _make_pass2_pair()
_P1_HEAD = _make_pass1_head()
_P2_HEAD = _make_pass2_head()


def kernel(x, edge_index, batch, global_feat, Wl1, bl1, Wr1, br1, att1, bias1,
           Wl2, bl2, Wr2, br2, att2, bias2, gamma1, beta1, gamma2, beta2,
           fc1_w, fc1_b, fc2_w, fc2_b):
    src = edge_index[0].astype(jnp.int32)
    dst = edge_index[1].astype(jnp.int32)
    zrow = jnp.zeros((NP, 128), jnp.float32)

    xlt1, xrt1, m1 = _proj_pair(x, Wl1, bl1, Wr1, br1, att1)
    ex1, dd1 = _P1_PAIR(xlt1, xrt1, src, dst, att1.reshape(-1), m1, zrow)
    dn1 = _dncols(dd1)
    out1 = _P2_PAIR(xlt1, src, dst, ex1, zrow)
    out1 = _norm_pair(out1, dn1)
    h = out1.reshape(NC, N, 2, 64).transpose(1, 0, 2, 3).reshape(N, 256)
    h = _bn_relu(h, bias1, gamma1, beta1)

    xlt2, xrt2, m2 = _proj_head(h, Wl2, bl2, Wr2, br2, att2)
    ex2, dd2 = _P1_HEAD(xlt2, xrt2, src, dst, att2.reshape(-1), m2, zrow)
    dn2 = _dncols(dd2)
    out2 = _P2_HEAD(xlt2, src, dst, ex2, zrow)
    out2 = _norm_head(out2, dn2)
    h = out2.reshape(H, N, 128).transpose(1, 0, 2).reshape(N, 512)
    h = _bn_relu(h, bias2, gamma2, beta2)

    out = _pool_mlp(h, batch, global_feat, fc1_w, fc1_b, fc2_w, fc2_b)
    return out.reshape(G)


# speedup vs baseline: 27.2050x; 1.2732x over previous
import functools
import jax
import jax.numpy as jnp
from jax import lax
from jax.experimental import pallas as pl
from jax.experimental.pallas import tpu as pltpu
from jax.experimental.pallas import tpu_sc as plsc

N = 10000
E = 640000
H = 4
G = 64
NC = 2
NS = 16
LANES = 16
EPT = E // NS
B = 160
B2 = 320
IDXW = 80
Q = B // IDXW
Q2 = B2 // IDXW
CH = EPT // B
CH2 = EPT // B2
NP = 10240
DR = NP // 8
DRT = DR // NS
DUMPR = 1000
DUMPT = N // DUMPR


def _lane_shuffle(v, idx):
    return lax.gather(
        v, idx[:, None],
        dimension_numbers=lax.GatherDimensionNumbers(
            offset_dims=(), collapsed_slice_dims=(0,), start_index_map=(0,)),
        slice_sizes=(1,),
        mode=lax.GatherScatterMode.PROMISE_IN_BOUNDS)


def _proj_pair_body(x_ref, wl_ref, bl_ref, wr_ref, br_ref, att_ref,
                    xlt_ref, xrt_ref, m_ref):
    c = pl.program_id(0)
    x = x_ref[...]
    blv = bl_ref[pl.ds(c, 1), :]
    brv = br_ref[pl.ds(c, 1), :]
    xl = jnp.dot(x, wl_ref[...], preferred_element_type=jnp.float32) + blv
    xr = jnp.dot(x, wr_ref[...], preferred_element_type=jnp.float32) + brv
    xlt_ref[...] = xl
    xrt_ref[...] = xr
    a0 = jnp.abs(att_ref[pl.ds(2 * c, 1), :])
    a1 = jnp.abs(att_ref[pl.ds(2 * c + 1, 1), :])
    m0 = (jnp.max(jnp.sum(jnp.abs(xl[:, :64]) * a0, axis=1))
          + jnp.max(jnp.sum(jnp.abs(xr[:, :64]) * a0, axis=1)))
    m1 = (jnp.max(jnp.sum(jnp.abs(xl[:, 64:]) * a1, axis=1))
          + jnp.max(jnp.sum(jnp.abs(xr[:, 64:]) * a1, axis=1)))
    m_ref[...] = jnp.concatenate(
        [jnp.full((4, 128), m0, jnp.float32),
         jnp.full((4, 128), m1, jnp.float32)], axis=0)


def _proj_pair(x, Wl, bl, Wr, br, att):
    n, f = x.shape
    return pl.pallas_call(
        _proj_pair_body,
        grid=(NC,),
        in_specs=[
            pl.BlockSpec((n, f), lambda c: (0, 0)),
            pl.BlockSpec((f, 128), lambda c: (0, c)),
            pl.BlockSpec((NC, 128), lambda c: (0, 0)),
            pl.BlockSpec((f, 128), lambda c: (0, c)),
            pl.BlockSpec((NC, 128), lambda c: (0, 0)),
            pl.BlockSpec((H, 64), lambda c: (0, 0)),
        ],
        out_specs=[
            pl.BlockSpec((n, 128), lambda c: (c, 0)),
            pl.BlockSpec((n, 128), lambda c: (c, 0)),
            pl.BlockSpec((8, 128), lambda c: (c, 0)),
        ],
        out_shape=[
            jax.ShapeDtypeStruct((NC * n, 128), jnp.float32),
            jax.ShapeDtypeStruct((NC * n, 128), jnp.float32),
            jax.ShapeDtypeStruct((NC * 8, 128), jnp.float32),
        ],
    )(x, Wl, bl.reshape(NC, 128), Wr, br.reshape(NC, 128), att)


def _proj_head_body(x_ref, wl_ref, bl_ref, wr_ref, br_ref, att_ref,
                    xlt_ref, xrt_ref, m_ref):
    h = pl.program_id(0)
    x = x_ref[...]
    blv = bl_ref[pl.ds(h, 1), :]
    brv = br_ref[pl.ds(h, 1), :]
    xl = jnp.dot(x, wl_ref[...], preferred_element_type=jnp.float32) + blv
    xr = jnp.dot(x, wr_ref[...], preferred_element_type=jnp.float32) + brv
    xlt_ref[...] = xl
    xrt_ref[...] = xr
    aab = jnp.abs(att_ref[pl.ds(h, 1), :])
    m = (jnp.max(jnp.sum(jnp.abs(xl) * aab, axis=1))
         + jnp.max(jnp.sum(jnp.abs(xr) * aab, axis=1)))
    m_ref[...] = jnp.full((8, 128), m, jnp.float32)


def _proj_head(x, Wl, bl, Wr, br, att):
    n, f = x.shape
    C = 128
    wlh = Wl.reshape(f, H, C).transpose(1, 0, 2).reshape(H * f, C)
    wrh = Wr.reshape(f, H, C).transpose(1, 0, 2).reshape(H * f, C)
    return pl.pallas_call(
        _proj_head_body,
        grid=(H,),
        in_specs=[
            pl.BlockSpec((n, f), lambda h: (0, 0)),
            pl.BlockSpec((f, C), lambda h: (h, 0)),
            pl.BlockSpec((H, C), lambda h: (0, 0)),
            pl.BlockSpec((f, C), lambda h: (h, 0)),
            pl.BlockSpec((H, C), lambda h: (0, 0)),
            pl.BlockSpec((H, C), lambda h: (0, 0)),
        ],
        out_specs=[
            pl.BlockSpec((n, C), lambda h: (h, 0)),
            pl.BlockSpec((n, C), lambda h: (h, 0)),
            pl.BlockSpec((8, 128), lambda h: (h, 0)),
        ],
        out_shape=[
            jax.ShapeDtypeStruct((H * n, C), jnp.float32),
            jax.ShapeDtypeStruct((H * n, C), jnp.float32),
            jax.ShapeDtypeStruct((8 * H, 128), jnp.float32),
        ],
    )(x, wlh, bl.reshape(H, C), wrh, br.reshape(H, C), att)


def _dncols(denom_dump):
    d = denom_dump.reshape(NC, DR, 128)[:, :, :16].reshape(NC, NP, 2)
    return d[:, :N, :].transpose(1, 0, 2).reshape(N, H)


def _sel_col(dn, h):
    msk = (lax.broadcasted_iota(jnp.int32, dn.shape, 1) == h).astype(dn.dtype)
    return jnp.sum(dn * msk, axis=1, keepdims=True)


def _norm_pair_body(h_ref, dn_ref, o_ref):
    c = pl.program_id(0)
    hv = h_ref[...]
    dn = dn_ref[...]
    d0 = _sel_col(dn, 2 * c)
    d1 = _sel_col(dn, 2 * c + 1)
    o_ref[...] = jnp.concatenate(
        [hv[:, :64] / (d0 + 1e-16), hv[:, 64:] / (d1 + 1e-16)], axis=1)


def _norm_pair(out1, dn):
    return pl.pallas_call(
        _norm_pair_body,
        grid=(NC,),
        in_specs=[pl.BlockSpec((N, 128), lambda c: (c, 0)),
                  pl.BlockSpec((N, H), lambda c: (0, 0))],
        out_specs=pl.BlockSpec((N, 128), lambda c: (c, 0)),
        out_shape=jax.ShapeDtypeStruct((NC * N, 128), jnp.float32),
    )(out1, dn)


def _norm_head_body(h_ref, dn_ref, o_ref):
    h = pl.program_id(0)
    o_ref[...] = h_ref[...] / (_sel_col(dn_ref[...], h) + 1e-16)


def _norm_head(out2, dn):
    return pl.pallas_call(
        _norm_head_body,
        grid=(H,),
        in_specs=[pl.BlockSpec((N, 128), lambda h: (h, 0)),
                  pl.BlockSpec((N, H), lambda h: (0, 0))],
        out_specs=pl.BlockSpec((N, 128), lambda h: (h, 0)),
        out_shape=jax.ShapeDtypeStruct((H * N, 128), jnp.float32),
    )(out2, dn)


def _bn_relu_body(h_ref, bias_ref, g_ref, b_ref, o_ref):
    hv = h_ref[...] + bias_ref[...]
    mu = jnp.mean(hv, axis=0)
    var = jnp.mean((hv - mu) ** 2, axis=0)
    o_ref[...] = jnp.maximum(
        (hv - mu) / jnp.sqrt(var + 1e-5) * g_ref[...] + b_ref[...], 0.0)


def _bn_relu(h, bias, gamma, beta):
    n, k = h.shape
    return pl.pallas_call(
        _bn_relu_body,
        grid=(k // 128,),
        in_specs=[
            pl.BlockSpec((n, 128), lambda j: (0, j)),
            pl.BlockSpec((128,), lambda j: (j,)),
            pl.BlockSpec((128,), lambda j: (j,)),
            pl.BlockSpec((128,), lambda j: (j,)),
        ],
        out_specs=pl.BlockSpec((n, 128), lambda j: (0, j)),
        out_shape=jax.ShapeDtypeStruct(h.shape, jnp.float32),
    )(h, bias, gamma, beta)


def _final_body(h_ref, batch_ref, gf_ref, w1a_ref, w1b_ref, b1_ref, w2_ref,
                b2_ref, o_ref):
    hv = h_ref[...]
    bat = batch_ref[...]
    gids = lax.broadcasted_iota(jnp.int32, (G, N), 0)
    P = jnp.where(bat == gids, 1.0, 0.0)
    counts = jnp.sum(P, axis=1)
    sums = jnp.dot(P, hv, preferred_element_type=jnp.float32)
    pooled = sums / jnp.maximum(counts, 1.0)[:, None]
    z = (jnp.dot(pooled, w1a_ref[...], preferred_element_type=jnp.float32)
         + jnp.dot(gf_ref[...], w1b_ref[...], preferred_element_type=jnp.float32)
         + b1_ref[...])
    z = jnp.maximum(z, 0.0)
    o_ref[...] = jnp.dot(z, w2_ref[...], preferred_element_type=jnp.float32) + b2_ref[...]


def _pool_mlp(h, batch, global_feat, fc1_w, fc1_b, fc2_w, fc2_b):
    k = h.shape[1]
    return pl.pallas_call(
        _final_body,
        out_shape=jax.ShapeDtypeStruct((G, 1), jnp.float32),
    )(h, batch.reshape(1, N).astype(jnp.int32), global_feat,
      fc1_w[:k], fc1_w[k:], fc1_b, fc2_w, fc2_b)


def _sc_mesh():
    return plsc.VectorSubcoreMesh(core_axis_name="c", subcore_axis_name="s")


def _build_idx(idx2d, flat_v, off, shift=0):
    for q in range(idx2d.shape[0]):
        for k in range(IDXW // LANES):
            sl = pl.ds(q * IDXW + k * LANES, LANES)
            v = flat_v[sl]
            if shift:
                v = lax.shift_right_logical(v, shift)
            idx2d[q, pl.ds(k * LANES, LANES)] = v + off


def _gather_rows(table, idx2d, buf, sem):
    cps = [pltpu.async_copy(table.at[idx2d.at[q]],
                            buf.at[pl.ds(q * IDXW, IDXW)], sem)
           for q in range(idx2d.shape[0])]
    for cp in cps:
        cp.wait()


def _issue_gathers(table, idx2d, buf, sem):
    for q in range(idx2d.shape[0]):
        pltpu.async_copy(table.at[idx2d.at[q]],
                         buf.at[pl.ds(q * IDXW, IDXW)], sem)


def _wait_gathers(table, idx2d, buf, sem):
    for q in range(idx2d.shape[0]):
        pltpu.make_async_copy(table.at[idx2d.at[q]],
                              buf.at[pl.ds(q * IDXW, IDXW)], sem).wait()


def _prefetch(srch, dsth, base, sv, dv, gi, di, goff, xlt, xrt, xb, xrb, s1, s2):
    pltpu.sync_copy(srch.at[pl.ds(base, B)], sv)
    pltpu.sync_copy(dsth.at[pl.ds(base, B)], dv)
    _build_idx(gi, sv, goff)
    _build_idx(di, dv, goff)
    _issue_gathers(xlt, gi, xb, s1)
    _issue_gathers(xrt, di, xrb, s2)


def _edge_logits(xl_buf, xr_buf, att_v, m_v, ex_buf, att_off, m_row, col0, C):
    m16 = m_v[m_row, pl.ds(0, LANES)]
    attcs = [att_v[pl.ds(att_off + cb * LANES, LANES)]
             for cb in range(C // LANES)]
    lanes = lax.iota(jnp.int32, LANES)
    bfly = [lanes ^ (1 << kk) for kk in range(4)]

    def grp(g, _):
        lvec = jnp.zeros((LANES,), jnp.float32)
        for j in range(LANES):
            e = g * LANES + j
            acc = jnp.zeros((LANES,), jnp.float32)
            for cb in range(C // LANES):
                csl = pl.ds(col0 + cb * LANES, LANES)
                sv = xl_buf[e, csl] + xr_buf[e, csl]
                acc = acc + jnp.maximum(sv, 0.2 * sv) * attcs[cb]
            for p in bfly:
                acc = acc + _lane_shuffle(acc, p)
            lvec = jnp.where(lanes == j, acc, lvec)
        ex_buf[pl.ds(g * LANES, LANES)] = jnp.exp(lvec - m16)
        return 0

    lax.fori_loop(0, B // LANES, grp, 0)


def _scatter_denom(oh_buf, ex_bufs, dst_v, sidx, denom_sh, lanes):
    def oh(g, _):
        sl = pl.ds(g * LANES, LANES)
        exs = [b[sl] for b in ex_bufs]
        col0 = (dst_v[sl] & 7) * 2
        for j in range(LANES):
            cj = col0[j]
            v = jnp.zeros((LANES,), jnp.float32)
            for hl in range(len(ex_bufs)):
                v = jnp.where(lanes == cj + hl, exs[hl][j], v)
            oh_buf[g * LANES + j, pl.ds(0, LANES)] = v
        return 0

    lax.fori_loop(0, B // LANES, oh, 0)
    for q in range(Q):
        pltpu.sync_copy(oh_buf.at[pl.ds(q * IDXW, IDXW)],
                        denom_sh.at[sidx.at[q]], add=True)


def _make_pass1_pair():
    C = 64

    @functools.partial(
        pl.kernel,
        out_type=[
            jax.ShapeDtypeStruct((H * E,), jnp.float32),
            jax.ShapeDtypeStruct((NC * DR, 128), jnp.float32),
        ],
        mesh=_sc_mesh(),
        scratch_types=[
            pltpu.VMEM((2 * B,), jnp.int32),
            pltpu.VMEM((2 * B,), jnp.int32),
            pltpu.VMEM((2 * Q, IDXW), jnp.int32),
            pltpu.VMEM((2 * Q, IDXW), jnp.int32),
            pltpu.VMEM((Q, IDXW), jnp.int32),
            pltpu.VMEM((2 * B, 128), jnp.float32),
            pltpu.VMEM((2 * B, 128), jnp.float32),
            pltpu.VMEM((B, 128), jnp.float32),
            pltpu.VMEM((B,), jnp.float32),
            pltpu.VMEM((B,), jnp.float32),
            pltpu.VMEM((H * C,), jnp.float32),
            pltpu.VMEM((NC * 8, 128), jnp.float32),
            pltpu.VMEM_SHARED((DR, 128), jnp.float32),
            pltpu.SemaphoreType.DMA,
            pltpu.SemaphoreType.DMA,
            pltpu.SemaphoreType.DMA,
            pltpu.SemaphoreType.DMA,
        ],
    )
    def p1(xlt, xrt, srch, dsth, atth, mh, zrowh, exh, denomdh,
           src_v, dst_v, gidx, didx, sidx, xl_buf, xr_buf, oh_buf, ex0, ex1,
           att_v, m_v, denom_sh, sem1a, sem2a, sem1b, sem2b):
        c = lax.axis_index("c")
        s = lax.axis_index("s")
        pltpu.sync_copy(atth, att_v)
        pltpu.sync_copy(mh, m_v)
        pltpu.sync_copy(zrowh.at[pl.ds(0, B)], oh_buf)
        pltpu.sync_copy(zrowh.at[pl.ds(s * DRT, DRT)],
                        denom_sh.at[pl.ds(s * DRT, DRT)])
        plsc.subcore_barrier()
        tile_base = s * EPT
        rowoff = c * N
        lanes = lax.iota(jnp.int32, LANES)
        sets = [
            (src_v.at[pl.ds(0, B)], dst_v.at[pl.ds(0, B)],
             gidx.at[pl.ds(0, Q)], didx.at[pl.ds(0, Q)],
             xl_buf.at[pl.ds(0, B)], xr_buf.at[pl.ds(0, B)], sem1a, sem2a),
            (src_v.at[pl.ds(B, B)], dst_v.at[pl.ds(B, B)],
             gidx.at[pl.ds(Q, Q)], didx.at[pl.ds(Q, Q)],
             xl_buf.at[pl.ds(B, B)], xr_buf.at[pl.ds(B, B)], sem1b, sem2b),
        ]

        def pf(setp, base):
            sv, dv, gi, di, xb, xrb, s1, s2 = setp
            _prefetch(srch, dsth, base, sv, dv, gi, di, rowoff,
                      xlt, xrt, xb, xrb, s1, s2)

        pf(sets[0], tile_base)

        def chunk_body(i, _):
            for p in range(2):
                sv, dv, gi, di, xb, xrb, s1, s2 = sets[p]
                cur = 2 * i + p
                base = tile_base + cur * B

                if p == 0:
                    pf(sets[1], base + B)
                else:
                    @pl.when(i < CH // 2 - 1)
                    def _():
                        pf(sets[0], base + B)

                _wait_gathers(xlt, gi, xb, s1)
                _wait_gathers(xrt, di, xrb, s2)
                for hl, exb in ((0, ex0), (1, ex1)):
                    hg = 2 * c + hl
                    _edge_logits(xb, xrb, att_v, m_v, exb,
                                 hg * C, c * 8 + hl * 4, hl * C, C)
                    pltpu.sync_copy(exb, exh.at[pl.ds(hg * E + base, B)])
                _build_idx(sidx, dv, 0, shift=3)
                _scatter_denom(oh_buf, (ex0, ex1), dv, sidx, denom_sh, lanes)
            return 0

        lax.fori_loop(0, CH // 2, chunk_body, 0)
        plsc.subcore_barrier()
        pltpu.sync_copy(denom_sh.at[pl.ds(s * DRT, DRT)],
                        denomdh.at[pl.ds(c * DR + s * DRT, DRT)])

    return p1


def _make_pass2_pair():
    C = 64

    @functools.partial(
        pl.kernel,
        out_type=jax.ShapeDtypeStruct((NC * N, 128), jnp.float32),
        mesh=_sc_mesh(),
        scratch_types=[
            pltpu.VMEM((B2,), jnp.int32),
            pltpu.VMEM((B2,), jnp.int32),
            pltpu.VMEM((Q2, IDXW), jnp.int32),
            pltpu.VMEM((Q2, IDXW), jnp.int32),
            pltpu.VMEM((B2, 128), jnp.float32),
            pltpu.VMEM((B2,), jnp.float32),
            pltpu.VMEM((B2,), jnp.float32),
            pltpu.VMEM_SHARED((N, 128), jnp.float32),
            pltpu.SemaphoreType.DMA,
        ],
    )
    def p2(xlt, srch, dsth, exh, zrowh, outh,
           src_v, dst_v, gidx, sidx, xl_buf, ex0, ex1, out_sh, sem):
        c = lax.axis_index("c")
        s = lax.axis_index("s")
        tile_base = s * EPT
        rowoff = c * N

        @pl.when(s < DUMPT)
        def _():
            pltpu.sync_copy(zrowh.at[pl.ds(s * DUMPR, DUMPR)],
                            out_sh.at[pl.ds(s * DUMPR, DUMPR)])

        plsc.subcore_barrier()

        def chunk_body(i, _):
            base = tile_base + i * B2
            pltpu.sync_copy(srch.at[pl.ds(base, B2)], src_v)
            pltpu.sync_copy(dsth.at[pl.ds(base, B2)], dst_v)
            _build_idx(gidx, src_v, rowoff)
            _build_idx(sidx, dst_v, 0)
            _gather_rows(xlt, gidx, xl_buf, sem)
            pltpu.sync_copy(exh.at[pl.ds((2 * c) * E + base, B2)], ex0)
            pltpu.sync_copy(exh.at[pl.ds((2 * c + 1) * E + base, B2)], ex1)

            def grp(g, _):
                sl = pl.ds(g * LANES, LANES)
                a0 = ex0[sl]
                a1 = ex1[sl]
                for j in range(LANES):
                    e = g * LANES + j
                    aj0 = a0[j]
                    aj1 = a1[j]
                    for cb in range(C // LANES):
                        csl = pl.ds(cb * LANES, LANES)
                        xl_buf[e, csl] = xl_buf[e, csl] * aj0
                    for cb in range(C // LANES):
                        csl = pl.ds(C + cb * LANES, LANES)
                        xl_buf[e, csl] = xl_buf[e, csl] * aj1
                return 0

            lax.fori_loop(0, B2 // LANES, grp, 0)
            for q in range(Q2):
                pltpu.sync_copy(xl_buf.at[pl.ds(q * IDXW, IDXW)],
                                out_sh.at[sidx.at[q]], add=True)
            return 0

        lax.fori_loop(0, CH2, chunk_body, 0)
        plsc.subcore_barrier()

        @pl.when(s < DUMPT)
        def _():
            pltpu.sync_copy(out_sh.at[pl.ds(s * DUMPR, DUMPR)],
                            outh.at[pl.ds(rowoff + s * DUMPR, DUMPR)])

    return p2


def _make_pass1_head():
    C = 128

    @functools.partial(
        pl.kernel,
        out_type=[
            jax.ShapeDtypeStruct((H * E,), jnp.float32),
            jax.ShapeDtypeStruct((NC * DR, 128), jnp.float32),
        ],
        mesh=_sc_mesh(),
        scratch_types=[
            pltpu.VMEM((2 * B,), jnp.int32),
            pltpu.VMEM((2 * B,), jnp.int32),
            pltpu.VMEM((2 * Q, IDXW), jnp.int32),
            pltpu.VMEM((2 * Q, IDXW), jnp.int32),
            pltpu.VMEM((Q, IDXW), jnp.int32),
            pltpu.VMEM((2 * B, 128), jnp.float32),
            pltpu.VMEM((2 * B, 128), jnp.float32),
            pltpu.VMEM((B, 128), jnp.float32),
            pltpu.VMEM((B,), jnp.float32),
            pltpu.VMEM((H * C,), jnp.float32),
            pltpu.VMEM((8 * H, 128), jnp.float32),
            pltpu.VMEM_SHARED((DR, 128), jnp.float32),
            pltpu.SemaphoreType.DMA,
            pltpu.SemaphoreType.DMA,
            pltpu.SemaphoreType.DMA,
            pltpu.SemaphoreType.DMA,
        ],
    )
    def p1(xlt, xrt, srch, dsth, atth, mh, zrowh, exh, denomdh,
           src_v, dst_v, gidx, didx, sidx, xl_buf, xr_buf, oh_buf, ex_buf,
           att_v, m_v, denom_sh, sem1a, sem2a, sem1b, sem2b):
        c = lax.axis_index("c")
        s = lax.axis_index("s")
        pltpu.sync_copy(atth, att_v)
        pltpu.sync_copy(mh, m_v)
        pltpu.sync_copy(zrowh.at[pl.ds(0, B)], oh_buf)
        pltpu.sync_copy(zrowh.at[pl.ds(s * DRT, DRT)],
                        denom_sh.at[pl.ds(s * DRT, DRT)])
        plsc.subcore_barrier()
        tile_base = s * EPT
        lanes = lax.iota(jnp.int32, LANES)
        sets = [
            (src_v.at[pl.ds(0, B)], dst_v.at[pl.ds(0, B)],
             gidx.at[pl.ds(0, Q)], didx.at[pl.ds(0, Q)],
             xl_buf.at[pl.ds(0, B)], xr_buf.at[pl.ds(0, B)], sem1a, sem2a),
            (src_v.at[pl.ds(B, B)], dst_v.at[pl.ds(B, B)],
             gidx.at[pl.ds(Q, Q)], didx.at[pl.ds(Q, Q)],
             xl_buf.at[pl.ds(B, B)], xr_buf.at[pl.ds(B, B)], sem1b, sem2b),
        ]

        for hl in range(2):
            hg = 2 * c + hl
            hoff = hg * N

            def pf(setp, base, hoff=hoff):
                sv, dv, gi, di, xb, xrb, s1, s2 = setp
                _prefetch(srch, dsth, base, sv, dv, gi, di, hoff,
                          xlt, xrt, xb, xrb, s1, s2)

            pf(sets[0], tile_base)

            def chunk_body(i, _, hl=hl, hg=hg, pf=pf):
                for p in range(2):
                    sv, dv, gi, di, xb, xrb, s1, s2 = sets[p]
                    cur = 2 * i + p
                    base = tile_base + cur * B

                    if p == 0:
                        pf(sets[1], base + B)
                    else:
                        @pl.when(i < CH // 2 - 1)
                        def _():
                            pf(sets[0], base + B)

                    _wait_gathers(xlt, gi, xb, s1)
                    _wait_gathers(xrt, di, xrb, s2)
                    _edge_logits(xb, xrb, att_v, m_v, ex_buf,
                                 hg * C, hg * 8, 0, C)
                    pltpu.sync_copy(ex_buf, exh.at[pl.ds(hg * E + base, B)])
                    _build_idx(sidx, dv, 0, shift=3)

                    def oh(g, _, dv=dv):
                        sl = pl.ds(g * LANES, LANES)
                        exs = ex_buf[sl]
                        col16 = (dv[sl] & 7) * 2 + hl
                        for j in range(LANES):
                            v = jnp.where(lanes == col16[j], exs[j], 0.0)
                            oh_buf[g * LANES + j, pl.ds(0, LANES)] = v
                        return 0

                    lax.fori_loop(0, B // LANES, oh, 0)
                    for q in range(Q):
                        pltpu.sync_copy(oh_buf.at[pl.ds(q * IDXW, IDXW)],
                                        denom_sh.at[sidx.at[q]], add=True)
                return 0

            lax.fori_loop(0, CH // 2, chunk_body, 0)

        plsc.subcore_barrier()
        pltpu.sync_copy(denom_sh.at[pl.ds(s * DRT, DRT)],
                        denomdh.at[pl.ds(c * DR + s * DRT, DRT)])

    return p1


def _make_pass2_head():
    C = 128

    @functools.partial(
        pl.kernel,
        out_type=jax.ShapeDtypeStruct((H * N, 128), jnp.float32),
        mesh=_sc_mesh(),
        scratch_types=[
            pltpu.VMEM((B2,), jnp.int32),
            pltpu.VMEM((B2,), jnp.int32),
            pltpu.VMEM((Q2, IDXW), jnp.int32),
            pltpu.VMEM((Q2, IDXW), jnp.int32),
            pltpu.VMEM((B2, 128), jnp.float32),
            pltpu.VMEM((B2,), jnp.float32),
            pltpu.VMEM_SHARED((N, 128), jnp.float32),
            pltpu.SemaphoreType.DMA,
        ],
    )
    def p2(xlt, srch, dsth, exh, zrowh, outh,
           src_v, dst_v, gidx, sidx, xl_buf, ex_buf, out_sh, sem):
        c = lax.axis_index("c")
        s = lax.axis_index("s")
        tile_base = s * EPT
        for hl in range(2):
            hg = 2 * c + hl
            hoff = hg * N

            @pl.when(s < DUMPT)
            def _():
                pltpu.sync_copy(zrowh.at[pl.ds(s * DUMPR, DUMPR)],
                                out_sh.at[pl.ds(s * DUMPR, DUMPR)])

            plsc.subcore_barrier()

            def chunk_body(i, _, hg=hg, hoff=hoff):
                base = tile_base + i * B2
                pltpu.sync_copy(srch.at[pl.ds(base, B2)], src_v)
                pltpu.sync_copy(dsth.at[pl.ds(base, B2)], dst_v)
                _build_idx(gidx, src_v, hoff)
                _build_idx(sidx, dst_v, 0)
                _gather_rows(xlt, gidx, xl_buf, sem)
                pltpu.sync_copy(exh.at[pl.ds(hg * E + base, B2)], ex_buf)

                def grp(g, _):
                    sl = pl.ds(g * LANES, LANES)
                    a16 = ex_buf[sl]
                    for j in range(LANES):
                        e = g * LANES + j
                        aj = a16[j]
                        for cb in range(C // LANES):
                            csl = pl.ds(cb * LANES, LANES)
                            xl_buf[e, csl] = xl_buf[e, csl] * aj
                    return 0

                lax.fori_loop(0, B2 // LANES, grp, 0)
                for q in range(Q2):
                    pltpu.sync_copy(xl_buf.at[pl.ds(q * IDXW, IDXW)],
                                    out_sh.at[sidx.at[q]], add=True)
                return 0

            lax.fori_loop(0, CH2, chunk_body, 0)
            plsc.subcore_barrier()

            @pl.when(s < DUMPT)
            def _(hoff=hoff):
                pltpu.sync_copy(out_sh.at[pl.ds(s * DUMPR, DUMPR)],
                                outh.at[pl.ds(hoff + s * DUMPR, DUMPR)])

            plsc.subcore_barrier()

    return p2


_P1_PAIR = _make_pass1_pair()
_P2_PAIR = _make_pass2_pair()
_P1_HEAD = _make_pass1_head()
_P2_HEAD = _make_pass2_head()


def kernel(x, edge_index, batch, global_feat, Wl1, bl1, Wr1, br1, att1, bias1,
           Wl2, bl2, Wr2, br2, att2, bias2, gamma1, beta1, gamma2, beta2,
           fc1_w, fc1_b, fc2_w, fc2_b):
    src = edge_index[0].astype(jnp.int32)
    dst = edge_index[1].astype(jnp.int32)
    zrow = jnp.zeros((NP, 128), jnp.float32)

    xlt1, xrt1, m1 = _proj_pair(x, Wl1, bl1, Wr1, br1, att1)
    ex1, dd1 = _P1_PAIR(xlt1, xrt1, src, dst, att1.reshape(-1), m1, zrow)
    dn1 = _dncols(dd1)
    out1 = _P2_PAIR(xlt1, src, dst, ex1, zrow)
    out1 = _norm_pair(out1, dn1)
    h = out1.reshape(NC, N, 2, 64).transpose(1, 0, 2, 3).reshape(N, 256)
    h = _bn_relu(h, bias1, gamma1, beta1)

    xlt2, xrt2, m2 = _proj_head(h, Wl2, bl2, Wr2, br2, att2)
    ex2, dd2 = _P1_HEAD(xlt2, xrt2, src, dst, att2.reshape(-1), m2, zrow)
    dn2 = _dncols(dd2)
    out2 = _P2_HEAD(xlt2, src, dst, ex2, zrow)
    out2 = _norm_head(out2, dn2)
    h = out2.reshape(H, N, 128).transpose(1, 0, 2).reshape(N, 512)
    h = _bn_relu(h, bias2, gamma2, beta2)

    out = _pool_mlp(h, batch, global_feat, fc1_w, fc1_b, fc2_w, fc2_b)
    return out.reshape(G)


# speedup vs baseline: 30.7924x; 1.1319x over previous
import functools
import jax
import jax.numpy as jnp
from jax import lax
from jax.experimental import pallas as pl
from jax.experimental.pallas import tpu as pltpu
from jax.experimental.pallas import tpu_sc as plsc

N = 10000
E = 640000
H = 4
G = 64
NC = 2
NS = 16
LANES = 16
EPT = E // NS
B = 160
B2 = 160
IDXW = 80
Q = B // IDXW
Q2 = B2 // IDXW
CH = EPT // B
CH2 = EPT // B2
NP = 10240
DR = NP // 8
DRT = DR // NS
DUMPR = 1000
DUMPT = N // DUMPR


def _lane_shuffle(v, idx):
    return lax.gather(
        v, idx[:, None],
        dimension_numbers=lax.GatherDimensionNumbers(
            offset_dims=(), collapsed_slice_dims=(0,), start_index_map=(0,)),
        slice_sizes=(1,),
        mode=lax.GatherScatterMode.PROMISE_IN_BOUNDS)


def _proj_pair_body(x_ref, wl_ref, bl_ref, wr_ref, br_ref, att_ref,
                    xlt_ref, xrt_ref, m_ref):
    c = pl.program_id(0)
    x = x_ref[...]
    blv = bl_ref[pl.ds(c, 1), :]
    brv = br_ref[pl.ds(c, 1), :]
    xl = jnp.dot(x, wl_ref[...], preferred_element_type=jnp.float32) + blv
    xr = jnp.dot(x, wr_ref[...], preferred_element_type=jnp.float32) + brv
    xlt_ref[...] = xl
    xrt_ref[...] = xr
    a0 = jnp.abs(att_ref[pl.ds(2 * c, 1), :])
    a1 = jnp.abs(att_ref[pl.ds(2 * c + 1, 1), :])
    m0 = (jnp.max(jnp.sum(jnp.abs(xl[:, :64]) * a0, axis=1))
          + jnp.max(jnp.sum(jnp.abs(xr[:, :64]) * a0, axis=1)))
    m1 = (jnp.max(jnp.sum(jnp.abs(xl[:, 64:]) * a1, axis=1))
          + jnp.max(jnp.sum(jnp.abs(xr[:, 64:]) * a1, axis=1)))
    m_ref[...] = jnp.concatenate(
        [jnp.full((4, 128), m0, jnp.float32),
         jnp.full((4, 128), m1, jnp.float32)], axis=0)


def _proj_pair(x, Wl, bl, Wr, br, att):
    n, f = x.shape
    return pl.pallas_call(
        _proj_pair_body,
        grid=(NC,),
        in_specs=[
            pl.BlockSpec((n, f), lambda c: (0, 0)),
            pl.BlockSpec((f, 128), lambda c: (0, c)),
            pl.BlockSpec((NC, 128), lambda c: (0, 0)),
            pl.BlockSpec((f, 128), lambda c: (0, c)),
            pl.BlockSpec((NC, 128), lambda c: (0, 0)),
            pl.BlockSpec((H, 64), lambda c: (0, 0)),
        ],
        out_specs=[
            pl.BlockSpec((n, 128), lambda c: (c, 0)),
            pl.BlockSpec((n, 128), lambda c: (c, 0)),
            pl.BlockSpec((8, 128), lambda c: (c, 0)),
        ],
        out_shape=[
            jax.ShapeDtypeStruct((NC * n, 128), jnp.float32),
            jax.ShapeDtypeStruct((NC * n, 128), jnp.float32),
            jax.ShapeDtypeStruct((NC * 8, 128), jnp.float32),
        ],
    )(x, Wl, bl.reshape(NC, 128), Wr, br.reshape(NC, 128), att)


def _proj_head_body(x_ref, wl_ref, bl_ref, wr_ref, br_ref, att_ref,
                    xlt_ref, xrt_ref, m_ref):
    h = pl.program_id(0)
    x = x_ref[...]
    blv = bl_ref[pl.ds(h, 1), :]
    brv = br_ref[pl.ds(h, 1), :]
    xl = jnp.dot(x, wl_ref[...], preferred_element_type=jnp.float32) + blv
    xr = jnp.dot(x, wr_ref[...], preferred_element_type=jnp.float32) + brv
    xlt_ref[...] = xl
    xrt_ref[...] = xr
    aab = jnp.abs(att_ref[pl.ds(h, 1), :])
    m = (jnp.max(jnp.sum(jnp.abs(xl) * aab, axis=1))
         + jnp.max(jnp.sum(jnp.abs(xr) * aab, axis=1)))
    m_ref[...] = jnp.full((8, 128), m, jnp.float32)


def _proj_head(x, Wl, bl, Wr, br, att):
    n, f = x.shape
    C = 128
    wlh = Wl.reshape(f, H, C).transpose(1, 0, 2).reshape(H * f, C)
    wrh = Wr.reshape(f, H, C).transpose(1, 0, 2).reshape(H * f, C)
    return pl.pallas_call(
        _proj_head_body,
        grid=(H,),
        in_specs=[
            pl.BlockSpec((n, f), lambda h: (0, 0)),
            pl.BlockSpec((f, C), lambda h: (h, 0)),
            pl.BlockSpec((H, C), lambda h: (0, 0)),
            pl.BlockSpec((f, C), lambda h: (h, 0)),
            pl.BlockSpec((H, C), lambda h: (0, 0)),
            pl.BlockSpec((H, C), lambda h: (0, 0)),
        ],
        out_specs=[
            pl.BlockSpec((n, C), lambda h: (h, 0)),
            pl.BlockSpec((n, C), lambda h: (h, 0)),
            pl.BlockSpec((8, 128), lambda h: (h, 0)),
        ],
        out_shape=[
            jax.ShapeDtypeStruct((H * n, C), jnp.float32),
            jax.ShapeDtypeStruct((H * n, C), jnp.float32),
            jax.ShapeDtypeStruct((8 * H, 128), jnp.float32),
        ],
    )(x, wlh, bl.reshape(H, C), wrh, br.reshape(H, C), att)


def _dncols(denom_dump):
    d = denom_dump.reshape(NC, DR, 128)[:, :, :16].reshape(NC, NP, 2)
    return d[:, :N, :].transpose(1, 0, 2).reshape(N, H)


def _sel_col(dn, h):
    msk = (lax.broadcasted_iota(jnp.int32, dn.shape, 1) == h).astype(dn.dtype)
    return jnp.sum(dn * msk, axis=1, keepdims=True)


def _norm_pair_body(h_ref, dn_ref, o_ref):
    c = pl.program_id(0)
    hv = h_ref[...]
    dn = dn_ref[...]
    d0 = _sel_col(dn, 2 * c)
    d1 = _sel_col(dn, 2 * c + 1)
    o_ref[...] = jnp.concatenate(
        [hv[:, :64] / (d0 + 1e-16), hv[:, 64:] / (d1 + 1e-16)], axis=1)


def _norm_pair(out1, dn):
    return pl.pallas_call(
        _norm_pair_body,
        grid=(NC,),
        in_specs=[pl.BlockSpec((N, 128), lambda c: (c, 0)),
                  pl.BlockSpec((N, H), lambda c: (0, 0))],
        out_specs=pl.BlockSpec((N, 128), lambda c: (c, 0)),
        out_shape=jax.ShapeDtypeStruct((NC * N, 128), jnp.float32),
    )(out1, dn)


def _norm_head_body(h_ref, dn_ref, o_ref):
    h = pl.program_id(0)
    o_ref[...] = h_ref[...] / (_sel_col(dn_ref[...], h) + 1e-16)


def _norm_head(out2, dn):
    return pl.pallas_call(
        _norm_head_body,
        grid=(H,),
        in_specs=[pl.BlockSpec((N, 128), lambda h: (h, 0)),
                  pl.BlockSpec((N, H), lambda h: (0, 0))],
        out_specs=pl.BlockSpec((N, 128), lambda h: (h, 0)),
        out_shape=jax.ShapeDtypeStruct((H * N, 128), jnp.float32),
    )(out2, dn)


def _bn_relu_body(h_ref, bias_ref, g_ref, b_ref, o_ref):
    hv = h_ref[...] + bias_ref[...]
    mu = jnp.mean(hv, axis=0)
    var = jnp.mean((hv - mu) ** 2, axis=0)
    o_ref[...] = jnp.maximum(
        (hv - mu) / jnp.sqrt(var + 1e-5) * g_ref[...] + b_ref[...], 0.0)


def _bn_relu(h, bias, gamma, beta):
    n, k = h.shape
    return pl.pallas_call(
        _bn_relu_body,
        grid=(k // 128,),
        in_specs=[
            pl.BlockSpec((n, 128), lambda j: (0, j)),
            pl.BlockSpec((128,), lambda j: (j,)),
            pl.BlockSpec((128,), lambda j: (j,)),
            pl.BlockSpec((128,), lambda j: (j,)),
        ],
        out_specs=pl.BlockSpec((n, 128), lambda j: (0, j)),
        out_shape=jax.ShapeDtypeStruct(h.shape, jnp.float32),
    )(h, bias, gamma, beta)


def _final_body(h_ref, batch_ref, gf_ref, w1a_ref, w1b_ref, b1_ref, w2_ref,
                b2_ref, o_ref):
    hv = h_ref[...]
    bat = batch_ref[...]
    gids = lax.broadcasted_iota(jnp.int32, (G, N), 0)
    P = jnp.where(bat == gids, 1.0, 0.0)
    counts = jnp.sum(P, axis=1)
    sums = jnp.dot(P, hv, preferred_element_type=jnp.float32)
    pooled = sums / jnp.maximum(counts, 1.0)[:, None]
    z = (jnp.dot(pooled, w1a_ref[...], preferred_element_type=jnp.float32)
         + jnp.dot(gf_ref[...], w1b_ref[...], preferred_element_type=jnp.float32)
         + b1_ref[...])
    z = jnp.maximum(z, 0.0)
    o_ref[...] = jnp.dot(z, w2_ref[...], preferred_element_type=jnp.float32) + b2_ref[...]


def _pool_mlp(h, batch, global_feat, fc1_w, fc1_b, fc2_w, fc2_b):
    k = h.shape[1]
    return pl.pallas_call(
        _final_body,
        out_shape=jax.ShapeDtypeStruct((G, 1), jnp.float32),
    )(h, batch.reshape(1, N).astype(jnp.int32), global_feat,
      fc1_w[:k], fc1_w[k:], fc1_b, fc2_w, fc2_b)


def _sc_mesh():
    return plsc.VectorSubcoreMesh(core_axis_name="c", subcore_axis_name="s")


def _build_idx(idx2d, flat_v, off, shift=0):
    for q in range(idx2d.shape[0]):
        for k in range(IDXW // LANES):
            sl = pl.ds(q * IDXW + k * LANES, LANES)
            v = flat_v[sl]
            if shift:
                v = lax.shift_right_logical(v, shift)
            idx2d[q, pl.ds(k * LANES, LANES)] = v + off


def _gather_rows(table, idx2d, buf, sem):
    cps = [pltpu.async_copy(table.at[idx2d.at[q]],
                            buf.at[pl.ds(q * IDXW, IDXW)], sem)
           for q in range(idx2d.shape[0])]
    for cp in cps:
        cp.wait()


def _issue_gathers(table, idx2d, buf, sem):
    for q in range(idx2d.shape[0]):
        pltpu.async_copy(table.at[idx2d.at[q]],
                         buf.at[pl.ds(q * IDXW, IDXW)], sem)


def _wait_gathers(table, idx2d, buf, sem):
    for q in range(idx2d.shape[0]):
        pltpu.make_async_copy(table.at[idx2d.at[q]],
                              buf.at[pl.ds(q * IDXW, IDXW)], sem).wait()


def _prefetch(srch, dsth, base, sv, dv, gi, di, goff, xlt, xrt, xb, xrb, s1, s2):
    pltpu.sync_copy(srch.at[pl.ds(base, B)], sv)
    pltpu.sync_copy(dsth.at[pl.ds(base, B)], dv)
    _build_idx(gi, sv, goff)
    _build_idx(di, dv, goff)
    _issue_gathers(xlt, gi, xb, s1)
    _issue_gathers(xrt, di, xrb, s2)


def _edge_logits(xl_buf, xr_buf, att_v, m_v, ex_buf, att_off, m_row, col0, C):
    m16 = m_v[m_row, pl.ds(0, LANES)]
    attcs = [att_v[pl.ds(att_off + cb * LANES, LANES)]
             for cb in range(C // LANES)]
    lanes = lax.iota(jnp.int32, LANES)
    bfly = [lanes ^ (1 << kk) for kk in range(4)]

    def grp(g, _):
        lvec = jnp.zeros((LANES,), jnp.float32)
        for j in range(LANES):
            e = g * LANES + j
            acc = jnp.zeros((LANES,), jnp.float32)
            for cb in range(C // LANES):
                csl = pl.ds(col0 + cb * LANES, LANES)
                sv = xl_buf[e, csl] + xr_buf[e, csl]
                acc = acc + jnp.maximum(sv, 0.2 * sv) * attcs[cb]
            for p in bfly:
                acc = acc + _lane_shuffle(acc, p)
            lvec = jnp.where(lanes == j, acc, lvec)
        ex_buf[pl.ds(g * LANES, LANES)] = jnp.exp(lvec - m16)
        return 0

    lax.fori_loop(0, B // LANES, grp, 0)


def _scatter_denom(oh_buf, ex_bufs, dst_v, sidx, denom_sh, lanes):
    def oh(g, _):
        sl = pl.ds(g * LANES, LANES)
        exs = [b[sl] for b in ex_bufs]
        col0 = (dst_v[sl] & 7) * 2
        for j in range(LANES):
            cj = col0[j]
            v = jnp.zeros((LANES,), jnp.float32)
            for hl in range(len(ex_bufs)):
                v = jnp.where(lanes == cj + hl, exs[hl][j], v)
            oh_buf[g * LANES + j, pl.ds(0, LANES)] = v
        return 0

    lax.fori_loop(0, B // LANES, oh, 0)
    for q in range(Q):
        pltpu.sync_copy(oh_buf.at[pl.ds(q * IDXW, IDXW)],
                        denom_sh.at[sidx.at[q]], add=True)


def _make_pass1_pair():
    C = 64

    @functools.partial(
        pl.kernel,
        out_type=[
            jax.ShapeDtypeStruct((H * E,), jnp.float32),
            jax.ShapeDtypeStruct((NC * DR, 128), jnp.float32),
        ],
        mesh=_sc_mesh(),
        scratch_types=[
            pltpu.VMEM((2 * B,), jnp.int32),
            pltpu.VMEM((2 * B,), jnp.int32),
            pltpu.VMEM((2 * Q, IDXW), jnp.int32),
            pltpu.VMEM((2 * Q, IDXW), jnp.int32),
            pltpu.VMEM((Q, IDXW), jnp.int32),
            pltpu.VMEM((2 * B, 128), jnp.float32),
            pltpu.VMEM((2 * B, 128), jnp.float32),
            pltpu.VMEM((B, 128), jnp.float32),
            pltpu.VMEM((B,), jnp.float32),
            pltpu.VMEM((B,), jnp.float32),
            pltpu.VMEM((H * C,), jnp.float32),
            pltpu.VMEM((NC * 8, 128), jnp.float32),
            pltpu.VMEM_SHARED((DR, 128), jnp.float32),
            pltpu.SemaphoreType.DMA,
            pltpu.SemaphoreType.DMA,
            pltpu.SemaphoreType.DMA,
            pltpu.SemaphoreType.DMA,
        ],
    )
    def p1(xlt, xrt, srch, dsth, atth, mh, zrowh, exh, denomdh,
           src_v, dst_v, gidx, didx, sidx, xl_buf, xr_buf, oh_buf, ex0, ex1,
           att_v, m_v, denom_sh, sem1a, sem2a, sem1b, sem2b):
        c = lax.axis_index("c")
        s = lax.axis_index("s")
        pltpu.sync_copy(atth, att_v)
        pltpu.sync_copy(mh, m_v)
        pltpu.sync_copy(zrowh.at[pl.ds(0, B)], oh_buf)
        pltpu.sync_copy(zrowh.at[pl.ds(s * DRT, DRT)],
                        denom_sh.at[pl.ds(s * DRT, DRT)])
        plsc.subcore_barrier()
        tile_base = s * EPT
        rowoff = c * N
        lanes = lax.iota(jnp.int32, LANES)
        sets = [
            (src_v.at[pl.ds(0, B)], dst_v.at[pl.ds(0, B)],
             gidx.at[pl.ds(0, Q)], didx.at[pl.ds(0, Q)],
             xl_buf.at[pl.ds(0, B)], xr_buf.at[pl.ds(0, B)], sem1a, sem2a),
            (src_v.at[pl.ds(B, B)], dst_v.at[pl.ds(B, B)],
             gidx.at[pl.ds(Q, Q)], didx.at[pl.ds(Q, Q)],
             xl_buf.at[pl.ds(B, B)], xr_buf.at[pl.ds(B, B)], sem1b, sem2b),
        ]

        def pf(setp, base):
            sv, dv, gi, di, xb, xrb, s1, s2 = setp
            _prefetch(srch, dsth, base, sv, dv, gi, di, rowoff,
                      xlt, xrt, xb, xrb, s1, s2)

        pf(sets[0], tile_base)

        def chunk_body(i, _):
            for p in range(2):
                sv, dv, gi, di, xb, xrb, s1, s2 = sets[p]
                cur = 2 * i + p
                base = tile_base + cur * B

                if p == 0:
                    pf(sets[1], base + B)
                else:
                    @pl.when(i < CH // 2 - 1)
                    def _():
                        pf(sets[0], base + B)

                _wait_gathers(xlt, gi, xb, s1)
                _wait_gathers(xrt, di, xrb, s2)
                for hl, exb in ((0, ex0), (1, ex1)):
                    hg = 2 * c + hl
                    _edge_logits(xb, xrb, att_v, m_v, exb,
                                 hg * C, c * 8 + hl * 4, hl * C, C)
                    pltpu.sync_copy(exb, exh.at[pl.ds(hg * E + base, B)])
                _build_idx(sidx, dv, 0, shift=3)
                _scatter_denom(oh_buf, (ex0, ex1), dv, sidx, denom_sh, lanes)
            return 0

        lax.fori_loop(0, CH // 2, chunk_body, 0)
        plsc.subcore_barrier()
        pltpu.sync_copy(denom_sh.at[pl.ds(s * DRT, DRT)],
                        denomdh.at[pl.ds(c * DR + s * DRT, DRT)])

    return p1


def _make_pass2_pair():
    C = 64

    @functools.partial(
        pl.kernel,
        out_type=jax.ShapeDtypeStruct((NC * N, 128), jnp.float32),
        mesh=_sc_mesh(),
        scratch_types=[
            pltpu.VMEM((2 * B2,), jnp.int32),
            pltpu.VMEM((2 * B2,), jnp.int32),
            pltpu.VMEM((2 * Q2, IDXW), jnp.int32),
            pltpu.VMEM((Q2, IDXW), jnp.int32),
            pltpu.VMEM((2 * B2, 128), jnp.float32),
            pltpu.VMEM((2 * B2,), jnp.float32),
            pltpu.VMEM((2 * B2,), jnp.float32),
            pltpu.VMEM_SHARED((N, 128), jnp.float32),
            pltpu.SemaphoreType.DMA,
            pltpu.SemaphoreType.DMA,
            pltpu.SemaphoreType.DMA,
            pltpu.SemaphoreType.DMA,
        ],
    )
    def p2(xlt, srch, dsth, exh, zrowh, outh,
           src_v, dst_v, gidx, sidx, xl_buf, ex0, ex1, out_sh,
           semga, semea, semgb, semeb):
        c = lax.axis_index("c")
        s = lax.axis_index("s")
        tile_base = s * EPT
        rowoff = c * N

        @pl.when(s < DUMPT)
        def _():
            pltpu.sync_copy(zrowh.at[pl.ds(s * DUMPR, DUMPR)],
                            out_sh.at[pl.ds(s * DUMPR, DUMPR)])

        plsc.subcore_barrier()
        sets = [
            (src_v.at[pl.ds(0, B2)], dst_v.at[pl.ds(0, B2)],
             gidx.at[pl.ds(0, Q2)], xl_buf.at[pl.ds(0, B2)],
             ex0.at[pl.ds(0, B2)], ex1.at[pl.ds(0, B2)], semga, semea),
            (src_v.at[pl.ds(B2, B2)], dst_v.at[pl.ds(B2, B2)],
             gidx.at[pl.ds(Q2, Q2)], xl_buf.at[pl.ds(B2, B2)],
             ex0.at[pl.ds(B2, B2)], ex1.at[pl.ds(B2, B2)], semgb, semeb),
        ]

        def pf(setp, base):
            sv, dv, gi, xb, e0, e1, sg, se = setp
            pltpu.sync_copy(srch.at[pl.ds(base, B2)], sv)
            pltpu.sync_copy(dsth.at[pl.ds(base, B2)], dv)
            _build_idx(gi, sv, rowoff)
            _issue_gathers(xlt, gi, xb, sg)
            pltpu.async_copy(exh.at[pl.ds((2 * c) * E + base, B2)], e0, se)
            pltpu.async_copy(exh.at[pl.ds((2 * c + 1) * E + base, B2)], e1, se)

        pf(sets[0], tile_base)

        def chunk_body(i, _):
            for p in range(2):
                sv, dv, gi, xb, e0, e1, sg, se = sets[p]
                cur = 2 * i + p
                base = tile_base + cur * B2

                if p == 0:
                    pf(sets[1], base + B2)
                else:
                    @pl.when(i < CH2 // 2 - 1)
                    def _():
                        pf(sets[0], base + B2)

                _wait_gathers(xlt, gi, xb, sg)
                pltpu.make_async_copy(
                    exh.at[pl.ds((2 * c) * E + base, B2)], e0, se).wait()
                pltpu.make_async_copy(
                    exh.at[pl.ds((2 * c + 1) * E + base, B2)], e1, se).wait()

                def grp(g, _, xb=xb, e0=e0, e1=e1):
                    sl = pl.ds(g * LANES, LANES)
                    a0 = e0[sl]
                    a1 = e1[sl]
                    for j in range(LANES):
                        e = g * LANES + j
                        aj0 = a0[j]
                        aj1 = a1[j]
                        for cb in range(C // LANES):
                            csl = pl.ds(cb * LANES, LANES)
                            xb[e, csl] = xb[e, csl] * aj0
                        for cb in range(C // LANES):
                            csl = pl.ds(C + cb * LANES, LANES)
                            xb[e, csl] = xb[e, csl] * aj1
                    return 0

                lax.fori_loop(0, B2 // LANES, grp, 0)
                _build_idx(sidx, dv, 0)
                for q in range(Q2):
                    pltpu.sync_copy(xb.at[pl.ds(q * IDXW, IDXW)],
                                    out_sh.at[sidx.at[q]], add=True)
            return 0

        lax.fori_loop(0, CH2 // 2, chunk_body, 0)
        plsc.subcore_barrier()

        @pl.when(s < DUMPT)
        def _():
            pltpu.sync_copy(out_sh.at[pl.ds(s * DUMPR, DUMPR)],
                            outh.at[pl.ds(rowoff + s * DUMPR, DUMPR)])

    return p2


def _make_pass1_head():
    C = 128

    @functools.partial(
        pl.kernel,
        out_type=[
            jax.ShapeDtypeStruct((H * E,), jnp.float32),
            jax.ShapeDtypeStruct((NC * DR, 128), jnp.float32),
        ],
        mesh=_sc_mesh(),
        scratch_types=[
            pltpu.VMEM((2 * B,), jnp.int32),
            pltpu.VMEM((2 * B,), jnp.int32),
            pltpu.VMEM((2 * Q, IDXW), jnp.int32),
            pltpu.VMEM((2 * Q, IDXW), jnp.int32),
            pltpu.VMEM((Q, IDXW), jnp.int32),
            pltpu.VMEM((2 * B, 128), jnp.float32),
            pltpu.VMEM((2 * B, 128), jnp.float32),
            pltpu.VMEM((B, 128), jnp.float32),
            pltpu.VMEM((B,), jnp.float32),
            pltpu.VMEM((H * C,), jnp.float32),
            pltpu.VMEM((8 * H, 128), jnp.float32),
            pltpu.VMEM_SHARED((DR, 128), jnp.float32),
            pltpu.SemaphoreType.DMA,
            pltpu.SemaphoreType.DMA,
            pltpu.SemaphoreType.DMA,
            pltpu.SemaphoreType.DMA,
        ],
    )
    def p1(xlt, xrt, srch, dsth, atth, mh, zrowh, exh, denomdh,
           src_v, dst_v, gidx, didx, sidx, xl_buf, xr_buf, oh_buf, ex_buf,
           att_v, m_v, denom_sh, sem1a, sem2a, sem1b, sem2b):
        c = lax.axis_index("c")
        s = lax.axis_index("s")
        pltpu.sync_copy(atth, att_v)
        pltpu.sync_copy(mh, m_v)
        pltpu.sync_copy(zrowh.at[pl.ds(0, B)], oh_buf)
        pltpu.sync_copy(zrowh.at[pl.ds(s * DRT, DRT)],
                        denom_sh.at[pl.ds(s * DRT, DRT)])
        plsc.subcore_barrier()
        tile_base = s * EPT
        lanes = lax.iota(jnp.int32, LANES)
        sets = [
            (src_v.at[pl.ds(0, B)], dst_v.at[pl.ds(0, B)],
             gidx.at[pl.ds(0, Q)], didx.at[pl.ds(0, Q)],
             xl_buf.at[pl.ds(0, B)], xr_buf.at[pl.ds(0, B)], sem1a, sem2a),
            (src_v.at[pl.ds(B, B)], dst_v.at[pl.ds(B, B)],
             gidx.at[pl.ds(Q, Q)], didx.at[pl.ds(Q, Q)],
             xl_buf.at[pl.ds(B, B)], xr_buf.at[pl.ds(B, B)], sem1b, sem2b),
        ]

        for hl in range(2):
            hg = 2 * c + hl
            hoff = hg * N

            def pf(setp, base, hoff=hoff):
                sv, dv, gi, di, xb, xrb, s1, s2 = setp
                _prefetch(srch, dsth, base, sv, dv, gi, di, hoff,
                          xlt, xrt, xb, xrb, s1, s2)

            pf(sets[0], tile_base)

            def chunk_body(i, _, hl=hl, hg=hg, pf=pf):
                for p in range(2):
                    sv, dv, gi, di, xb, xrb, s1, s2 = sets[p]
                    cur = 2 * i + p
                    base = tile_base + cur * B

                    if p == 0:
                        pf(sets[1], base + B)
                    else:
                        @pl.when(i < CH // 2 - 1)
                        def _():
                            pf(sets[0], base + B)

                    _wait_gathers(xlt, gi, xb, s1)
                    _wait_gathers(xrt, di, xrb, s2)
                    _edge_logits(xb, xrb, att_v, m_v, ex_buf,
                                 hg * C, hg * 8, 0, C)
                    pltpu.sync_copy(ex_buf, exh.at[pl.ds(hg * E + base, B)])
                    _build_idx(sidx, dv, 0, shift=3)

                    def oh(g, _, dv=dv):
                        sl = pl.ds(g * LANES, LANES)
                        exs = ex_buf[sl]
                        col16 = (dv[sl] & 7) * 2 + hl
                        for j in range(LANES):
                            v = jnp.where(lanes == col16[j], exs[j], 0.0)
                            oh_buf[g * LANES + j, pl.ds(0, LANES)] = v
                        return 0

                    lax.fori_loop(0, B // LANES, oh, 0)
                    for q in range(Q):
                        pltpu.sync_copy(oh_buf.at[pl.ds(q * IDXW, IDXW)],
                                        denom_sh.at[sidx.at[q]], add=True)
                return 0

            lax.fori_loop(0, CH // 2, chunk_body, 0)

        plsc.subcore_barrier()
        pltpu.sync_copy(denom_sh.at[pl.ds(s * DRT, DRT)],
                        denomdh.at[pl.ds(c * DR + s * DRT, DRT)])

    return p1


def _make_pass2_head():
    C = 128

    @functools.partial(
        pl.kernel,
        out_type=jax.ShapeDtypeStruct((H * N, 128), jnp.float32),
        mesh=_sc_mesh(),
        scratch_types=[
            pltpu.VMEM((2 * B2,), jnp.int32),
            pltpu.VMEM((2 * B2,), jnp.int32),
            pltpu.VMEM((2 * Q2, IDXW), jnp.int32),
            pltpu.VMEM((Q2, IDXW), jnp.int32),
            pltpu.VMEM((2 * B2, 128), jnp.float32),
            pltpu.VMEM((2 * B2,), jnp.float32),
            pltpu.VMEM_SHARED((N, 128), jnp.float32),
            pltpu.SemaphoreType.DMA,
            pltpu.SemaphoreType.DMA,
            pltpu.SemaphoreType.DMA,
            pltpu.SemaphoreType.DMA,
        ],
    )
    def p2(xlt, srch, dsth, exh, zrowh, outh,
           src_v, dst_v, gidx, sidx, xl_buf, ex_buf, out_sh,
           semga, semea, semgb, semeb):
        c = lax.axis_index("c")
        s = lax.axis_index("s")
        tile_base = s * EPT
        sets = [
            (src_v.at[pl.ds(0, B2)], dst_v.at[pl.ds(0, B2)],
             gidx.at[pl.ds(0, Q2)], xl_buf.at[pl.ds(0, B2)],
             ex_buf.at[pl.ds(0, B2)], semga, semea),
            (src_v.at[pl.ds(B2, B2)], dst_v.at[pl.ds(B2, B2)],
             gidx.at[pl.ds(Q2, Q2)], xl_buf.at[pl.ds(B2, B2)],
             ex_buf.at[pl.ds(B2, B2)], semgb, semeb),
        ]
        for hl in range(2):
            hg = 2 * c + hl
            hoff = hg * N

            @pl.when(s < DUMPT)
            def _():
                pltpu.sync_copy(zrowh.at[pl.ds(s * DUMPR, DUMPR)],
                                out_sh.at[pl.ds(s * DUMPR, DUMPR)])

            plsc.subcore_barrier()

            def pf(setp, base, hg=hg, hoff=hoff):
                sv, dv, gi, xb, eb, sg, se = setp
                pltpu.sync_copy(srch.at[pl.ds(base, B2)], sv)
                pltpu.sync_copy(dsth.at[pl.ds(base, B2)], dv)
                _build_idx(gi, sv, hoff)
                _issue_gathers(xlt, gi, xb, sg)
                pltpu.async_copy(exh.at[pl.ds(hg * E + base, B2)], eb, se)

            pf(sets[0], tile_base)

            def chunk_body(i, _, hg=hg, pf=pf):
                for p in range(2):
                    sv, dv, gi, xb, eb, sg, se = sets[p]
                    cur = 2 * i + p
                    base = tile_base + cur * B2

                    if p == 0:
                        pf(sets[1], base + B2)
                    else:
                        @pl.when(i < CH2 // 2 - 1)
                        def _():
                            pf(sets[0], base + B2)

                    _wait_gathers(xlt, gi, xb, sg)
                    pltpu.make_async_copy(
                        exh.at[pl.ds(hg * E + base, B2)], eb, se).wait()

                    def grp(g, _, xb=xb, eb=eb):
                        sl = pl.ds(g * LANES, LANES)
                        a16 = eb[sl]
                        for j in range(LANES):
                            e = g * LANES + j
                            aj = a16[j]
                            for cb in range(C // LANES):
                                csl = pl.ds(cb * LANES, LANES)
                                xb[e, csl] = xb[e, csl] * aj
                        return 0

                    lax.fori_loop(0, B2 // LANES, grp, 0)
                    _build_idx(sidx, dv, 0)
                    for q in range(Q2):
                        pltpu.sync_copy(xb.at[pl.ds(q * IDXW, IDXW)],
                                        out_sh.at[sidx.at[q]], add=True)
                return 0

            lax.fori_loop(0, CH2 // 2, chunk_body, 0)
            plsc.subcore_barrier()

            @pl.when(s < DUMPT)
            def _(hoff=hoff):
                pltpu.sync_copy(out_sh.at[pl.ds(s * DUMPR, DUMPR)],
                                outh.at[pl.ds(hoff + s * DUMPR, DUMPR)])

            plsc.subcore_barrier()

    return p2


_P1_PAIR = _make_pass1_pair()
_P2_PAIR = _make_pass2_pair()
_P1_HEAD = _make_pass1_head()
_P2_HEAD = _make_pass2_head()


def kernel(x, edge_index, batch, global_feat, Wl1, bl1, Wr1, br1, att1, bias1,
           Wl2, bl2, Wr2, br2, att2, bias2, gamma1, beta1, gamma2, beta2,
           fc1_w, fc1_b, fc2_w, fc2_b):
    src = edge_index[0].astype(jnp.int32)
    dst = edge_index[1].astype(jnp.int32)
    zrow = jnp.zeros((NP, 128), jnp.float32)

    xlt1, xrt1, m1 = _proj_pair(x, Wl1, bl1, Wr1, br1, att1)
    ex1, dd1 = _P1_PAIR(xlt1, xrt1, src, dst, att1.reshape(-1), m1, zrow)
    dn1 = _dncols(dd1)
    out1 = _P2_PAIR(xlt1, src, dst, ex1, zrow)
    out1 = _norm_pair(out1, dn1)
    h = out1.reshape(NC, N, 2, 64).transpose(1, 0, 2, 3).reshape(N, 256)
    h = _bn_relu(h, bias1, gamma1, beta1)

    xlt2, xrt2, m2 = _proj_head(h, Wl2, bl2, Wr2, br2, att2)
    ex2, dd2 = _P1_HEAD(xlt2, xrt2, src, dst, att2.reshape(-1), m2, zrow)
    dn2 = _dncols(dd2)
    out2 = _P2_HEAD(xlt2, src, dst, ex2, zrow)
    out2 = _norm_head(out2, dn2)
    h = out2.reshape(H, N, 128).transpose(1, 0, 2).reshape(N, 512)
    h = _bn_relu(h, bias2, gamma2, beta2)

    out = _pool_mlp(h, batch, global_feat, fc1_w, fc1_b, fc2_w, fc2_b)
    return out.reshape(G)
